# Initial kernel scaffold; baseline (speedup 1.0000x reference)
import jax
import jax.numpy as jnp
from jax.experimental import pallas as pl


def kernel(x, edge_index, W1_l, b1_l, W1_r, W2_l, b2_l, W2_r):
    raise NotImplementedError("write your pallas kernel here")


# speedup vs baseline: 1.8074x; 1.8074x over previous
import functools

import jax
import jax.numpy as jnp
from jax import lax
from jax.experimental import pallas as pl
from jax.experimental.pallas import tpu as pltpu
from jax.experimental.pallas import tpu_sc as plsc

N = 10000
E = 320000
D = 128
NC, NS, L = 2, 16, 16
NW = NC * NS
RPW = 313
NPAD = NW * RPW
SCAN_CH = 2000
FLUSH = 2048
STAGE = 4096
ECAP = E + FLUSH
GCH = 128
QD = D // L

_mesh = lambda: plsc.VectorSubcoreMesh(core_axis_name="c", subcore_axis_name="s")


def _gather_max_fold(h_hbm, srcp_hbm, ldstp_hbm, agg_hbm, idxv, ldv, rows2d,
                     agg1d, sem, wid, ct):
    iota = lax.iota(jnp.int32, L)
    neg = jnp.full((L,), -jnp.inf, dtype=jnp.float32)

    def init_body(i, _):
        for q in range(16):
            agg1d[pl.ds(i * 256 + q * L, L)] = neg
        return 0
    lax.fori_loop(0, (RPW + 1) * D // 256, init_body, 0)

    nch = (ct + GCH - 1) // GCH

    def chunk_body(g, _):
        base = pl.multiple_of(wid * ECAP + g * GCH, 8)
        pltpu.sync_copy(srcp_hbm.at[pl.ds(base, GCH)], idxv)
        pltpu.sync_copy(ldstp_hbm.at[pl.ds(base, GCH)], ldv)
        for q in range(GCH // L):
            pos = g * GCH + q * L + iota
            m = pos < ct
            idxv[pl.ds(q * L, L)] = jnp.where(m, idxv[pl.ds(q * L, L)], 0)
            ldv[pl.ds(q * L, L)] = jnp.where(m, ldv[pl.ds(q * L, L)], RPW)
        pltpu.async_copy(h_hbm.at[idxv], rows2d, sem).wait()

        def fold(e, _):
            esp = jnp.zeros((L,), jnp.int32) + e
            lds = plsc.load_gather(ldv, [esp])
            abase = lds * D
            for q in range(QD):
                col = q * L + iota
                addr = abase + col
                cur = plsc.load_gather(agg1d, [addr])
                r = plsc.load_gather(rows2d, [esp, col])
                plsc.store_scatter(agg1d, [addr], jnp.maximum(cur, r))
            return 0
        lax.fori_loop(0, GCH, fold, 0)
        return 0
    lax.fori_loop(0, nch, chunk_body, 0)

    pltpu.sync_copy(agg1d.at[pl.ds(0, RPW * D)],
                    agg_hbm.at[pl.ds(pl.multiple_of(wid * RPW * D, 8), RPW * D)])


def _sc_layer1_body(x_hbm, src_hbm, dst_hbm,
                    agg_hbm, srcp_hbm, ldstp_hbm, cnt_hbm,
                    dstv, srcv, stage_s, stage_d, idxv, ldv, rows2d, agg1d,
                    cbuf, sem):
    c = lax.axis_index("c")
    s = lax.axis_index("s")
    wid = c * NS + s
    lo = wid * RPW
    hi = jnp.minimum(lo + RPW, N)
    iota = lax.iota(jnp.int32, L)

    def chunk_body(ci, carry):
        vcnt, off = carry
        base = pl.multiple_of(ci * SCAN_CH, 8)
        pltpu.sync_copy(dst_hbm.at[pl.ds(base, SCAN_CH)], dstv)
        pltpu.sync_copy(src_hbm.at[pl.ds(base, SCAN_CH)], srcv)

        def grp(g, vc):
            d = dstv[pl.ds(g * L, L)]
            sv = srcv[pl.ds(g * L, L)]
            m = (d >= lo) & (d < hi)
            csum = jnp.cumsum(m.astype(jnp.int32))
            pos = vc + csum - 1
            plsc.store_scatter(stage_s, [pos], sv, mask=m)
            plsc.store_scatter(stage_d, [pos], d - lo, mask=m)
            return vc + jnp.max(csum)
        vcnt = lax.fori_loop(0, SCAN_CH // L, grp, vcnt)

        def do_flush(args):
            vc, o = args
            fo = pl.multiple_of(wid * ECAP + o, 8)
            pltpu.sync_copy(stage_s.at[pl.ds(0, FLUSH)],
                            srcp_hbm.at[pl.ds(fo, FLUSH)])
            pltpu.sync_copy(stage_d.at[pl.ds(0, FLUSH)],
                            ldstp_hbm.at[pl.ds(fo, FLUSH)])
            rem = vc - FLUSH

            def mv(i, _):
                stage_s[pl.ds(i * L, L)] = stage_s[pl.ds(FLUSH + i * L, L)]
                stage_d[pl.ds(i * L, L)] = stage_d[pl.ds(FLUSH + i * L, L)]
                return 0
            lax.fori_loop(0, (rem + L - 1) // L, mv, 0)
            return (rem, o + FLUSH)

        return lax.cond(vcnt >= FLUSH, do_flush, lambda a: a, (vcnt, off))

    vcnt, off = lax.fori_loop(0, E // SCAN_CH, chunk_body,
                              (jnp.int32(0), jnp.int32(0)))
    fo = pl.multiple_of(wid * ECAP + off, 8)
    pltpu.sync_copy(stage_s.at[pl.ds(0, FLUSH)], srcp_hbm.at[pl.ds(fo, FLUSH)])
    pltpu.sync_copy(stage_d.at[pl.ds(0, FLUSH)], ldstp_hbm.at[pl.ds(fo, FLUSH)])
    ct = off + vcnt
    cbuf[pl.ds(0, L)] = jnp.zeros((L,), jnp.int32) + ct
    pltpu.sync_copy(cbuf.at[pl.ds(0, L)],
                    cnt_hbm.at[pl.ds(pl.multiple_of(wid * L, 8), L)])

    _gather_max_fold(x_hbm, srcp_hbm, ldstp_hbm, agg_hbm, idxv, ldv, rows2d,
                     agg1d, sem, wid, ct)


def _sc_layer2_body(h_hbm, srcp_hbm, ldstp_hbm, cnt_hbm,
                    agg_hbm,
                    cntv, idxv, ldv, rows2d, agg1d, sem):
    c = lax.axis_index("c")
    s = lax.axis_index("s")
    wid = c * NS + s
    pltpu.sync_copy(cnt_hbm, cntv)
    ct = jnp.max(cntv[pl.ds(wid * L, L)])
    _gather_max_fold(h_hbm, srcp_hbm, ldstp_hbm, agg_hbm, idxv, ldv, rows2d,
                     agg1d, sem, wid, ct)


def _sc_layer1(x, src, dst):
    f = pl.kernel(
        _sc_layer1_body,
        out_type=[
            jax.ShapeDtypeStruct((NPAD * D,), jnp.float32),
            jax.ShapeDtypeStruct((NW * ECAP,), jnp.int32),
            jax.ShapeDtypeStruct((NW * ECAP,), jnp.int32),
            jax.ShapeDtypeStruct((NW * L,), jnp.int32),
        ],
        mesh=_mesh(),
        compiler_params=pltpu.CompilerParams(needs_layout_passes=False),
        scratch_types=[
            pltpu.VMEM((SCAN_CH,), jnp.int32),
            pltpu.VMEM((SCAN_CH,), jnp.int32),
            pltpu.VMEM((STAGE,), jnp.int32),
            pltpu.VMEM((STAGE,), jnp.int32),
            pltpu.VMEM((GCH,), jnp.int32),
            pltpu.VMEM((GCH,), jnp.int32),
            pltpu.VMEM((GCH, D), jnp.float32),
            pltpu.VMEM(((RPW + 1) * D,), jnp.float32),
            pltpu.VMEM((L,), jnp.int32),
            pltpu.SemaphoreType.DMA,
        ],
    )
    return f(x, src, dst)


def _sc_layer2(h, srcp, ldstp, cnt):
    f = pl.kernel(
        _sc_layer2_body,
        out_type=jax.ShapeDtypeStruct((NPAD * D,), jnp.float32),
        mesh=_mesh(),
        compiler_params=pltpu.CompilerParams(needs_layout_passes=False),
        scratch_types=[
            pltpu.VMEM((NW * L,), jnp.int32),
            pltpu.VMEM((GCH,), jnp.int32),
            pltpu.VMEM((GCH,), jnp.int32),
            pltpu.VMEM((GCH, D), jnp.float32),
            pltpu.VMEM(((RPW + 1) * D,), jnp.float32),
            pltpu.SemaphoreType.DMA,
        ],
    )
    return f(h, srcp, ldstp, cnt)


def _lin_body(relu, agg_ref, h_ref, wl_ref, wr_ref, b_ref, o_ref):
    a = agg_ref[...]
    a = jnp.where(a == -jnp.inf, 0.0, a)
    out = lax.dot_general(a, wl_ref[...], (((1,), (1,)), ((), ())),
                          preferred_element_type=jnp.float32)
    out = out + lax.dot_general(h_ref[...], wr_ref[...], (((1,), (1,)), ((), ())),
                                preferred_element_type=jnp.float32)
    out = out + b_ref[...]
    if relu:
        out = jnp.maximum(out, 0.0)
    o_ref[...] = out


def _linear(agg, h, W_l, b_l, W_r, relu):
    BM = 1000
    return pl.pallas_call(
        functools.partial(_lin_body, relu),
        grid=(N // BM,),
        in_specs=[
            pl.BlockSpec((BM, D), lambda i: (i, 0)),
            pl.BlockSpec((BM, D), lambda i: (i, 0)),
            pl.BlockSpec((D, D), lambda i: (0, 0)),
            pl.BlockSpec((D, D), lambda i: (0, 0)),
            pl.BlockSpec((1, D), lambda i: (0, 0)),
        ],
        out_specs=pl.BlockSpec((BM, D), lambda i: (i, 0)),
        out_shape=jax.ShapeDtypeStruct((N, D), jnp.float32),
    )(agg, h, W_l, W_r, b_l)


def kernel(x, edge_index, W1_l, b1_l, W1_r, W2_l, b2_l, W2_r):
    src = edge_index[0].astype(jnp.int32)
    dst = edge_index[1].astype(jnp.int32)
    agg1f, srcp, ldstp, cnt = _sc_layer1(x, src, dst)
    agg1 = agg1f.reshape(NPAD, D)[:N]
    h1 = _linear(agg1, x, W1_l, b1_l.reshape(1, D), W1_r, relu=True)
    agg2f = _sc_layer2(h1, srcp, ldstp, cnt)
    agg2 = agg2f.reshape(NPAD, D)[:N]
    return _linear(agg2, h1, W2_l, b2_l.reshape(1, D), W2_r, relu=False)


# speedup vs baseline: 2.5238x; 1.3964x over previous
import functools

import jax
import jax.numpy as jnp
from jax import lax
from jax.experimental import pallas as pl
from jax.experimental.pallas import tpu as pltpu
from jax.experimental.pallas import tpu_sc as plsc

N = 10000
E = 320000
D = 128
NC, NS, L = 2, 16, 16
NW = NC * NS
RPW = 313
NPAD = NW * RPW
SCAN_CH = 3200
FLUSH = 2048
STAGE = 4096 + 2048
ECAP = E + FLUSH
GB = 128
QD = D // L
SHIFT = 512

_mesh = lambda: plsc.VectorSubcoreMesh(core_axis_name="c", subcore_axis_name="s")


def _gather_max_fold(h_hbm, pk_hbm, agg_hbm, pkv, idxv, ldv, rows, agg1d,
                     sems, wid, ct):
    iota = lax.iota(jnp.int32, L)
    neg = jnp.full((L,), -jnp.inf, dtype=jnp.float32)

    def init_body(i, _):
        for q in range(16):
            agg1d[pl.ds(i * 256 + q * L, L)] = neg
        return 0
    lax.fori_loop(0, (RPW + 1) * D // 256, init_body, 0)

    nch = (ct + GB - 1) // GB

    def idx_start(g, b):
        base = pl.multiple_of(wid * ECAP + g * GB, 8)
        pltpu.async_copy(pk_hbm.at[pl.ds(base, GB)], pkv[b], sems[2 + b])

    def idx_wait_clean(g, b):
        pltpu.make_async_copy(pk_hbm.at[pl.ds(0, GB)], pkv[b],
                              sems[2 + b]).wait()
        for q in range(GB // L):
            w = pkv[b][pl.ds(q * L, L)]
            m = (g * GB + q * L + iota) < ct
            idxv[b][pl.ds(q * L, L)] = jnp.where(m, w // SHIFT, 0)
            ldv[b][pl.ds(q * L, L)] = jnp.where(m, w & (SHIFT - 1), RPW)

    def row_start(b):
        pltpu.async_copy(h_hbm.at[idxv[b]], rows[b], sems[b])

    def row_wait(b):
        pltpu.make_async_copy(h_hbm.at[pl.ds(0, GB)], rows[b], sems[b]).wait()

    def fold_chunk(b):
        def fold(e, _):
            esp = jnp.zeros((L,), jnp.int32) + e
            lds = plsc.load_gather(ldv[b], [esp])
            abase = lds * D
            for q in range(QD):
                col = q * L + iota
                addr = abase + col
                cur = plsc.load_gather(agg1d, [addr])
                r = plsc.load_gather(rows[b], [esp, col])
                plsc.store_scatter(agg1d, [addr], jnp.maximum(cur, r))
            return 0
        lax.fori_loop(0, GB, fold, 0)

    @pl.when(nch > 0)
    def _():
        idx_start(0, 0)
        idx_wait_clean(0, 0)
        row_start(0)

    @pl.when(nch > 1)
    def _():
        idx_start(1, 1)

    def pair(p, _):
        for b in range(2):
            g = p * 2 + b

            @pl.when(g < nch)
            def _():
                row_wait(b)

                @pl.when(g + 1 < nch)
                def _():
                    idx_wait_clean(g + 1, 1 - b)
                    row_start(1 - b)

                @pl.when(g + 2 < nch)
                def _():
                    idx_start(g + 2, b)

                fold_chunk(b)
        return 0
    lax.fori_loop(0, (nch + 1) // 2, pair, 0)

    pltpu.sync_copy(agg1d.at[pl.ds(0, RPW * D)],
                    agg_hbm.at[pl.ds(pl.multiple_of(wid * RPW * D, 8), RPW * D)])


def _sc_layer1_body(x_hbm, src_hbm, dst_hbm,
                    agg_hbm, pk_hbm, cnt_hbm,
                    dstv, srcv, stage, pkv, idxv, ldv, rows, agg1d,
                    cbuf, sem0, sem1, sem2, sem3, semd):
    c = lax.axis_index("c")
    s = lax.axis_index("s")
    wid = c * NS + s
    lo = wid * RPW
    hi = jnp.minimum(lo + RPW, N)
    iota = lax.iota(jnp.int32, L)

    def scan_start(ci, b):
        base = pl.multiple_of(ci * SCAN_CH, 8)
        pltpu.async_copy(dst_hbm.at[pl.ds(base, SCAN_CH)], dstv[b], semd)
        pltpu.async_copy(src_hbm.at[pl.ds(base, SCAN_CH)], srcv[b], semd)

    def scan_wait(b):
        pltpu.make_async_copy(dst_hbm.at[pl.ds(0, SCAN_CH)], dstv[b], semd).wait()
        pltpu.make_async_copy(src_hbm.at[pl.ds(0, SCAN_CH)], srcv[b], semd).wait()

    scan_start(0, 0)

    def chunk_one(ci, b, carry):
        vc, off = carry
        scan_wait(b)

        @pl.when(ci + 1 < E // SCAN_CH)
        def _():
            scan_start(ci + 1, 1 - b)

        def grp2(q, vcv):
            out = vcv
            for u in range(2):
                g = q * 2 + u
                d = dstv[b][pl.ds(g * L, L)]
                sv = srcv[b][pl.ds(g * L, L)]
                m = (d >= lo) & (d < hi)
                csum = jnp.cumsum(m.astype(jnp.int32))
                pos = out + csum - 1
                plsc.store_scatter(stage, [pos], sv * SHIFT + (d - lo), mask=m)
                out = out + plsc.all_reduce_population_count(m)
            return out
        vc = lax.fori_loop(0, SCAN_CH // L // 2, grp2, vc)
        vcs = jnp.max(vc)

        def do_flush(args):
            v, o = args
            k = vcs // FLUSH

            def fl(j, oo):
                so = pl.multiple_of(j * FLUSH, 8)
                fo = pl.multiple_of(wid * ECAP + oo, 8)
                pltpu.sync_copy(stage.at[pl.ds(so, FLUSH)],
                                pk_hbm.at[pl.ds(fo, FLUSH)])
                return oo + FLUSH
            o2 = lax.fori_loop(0, k, fl, o)
            rem = vcs - k * FLUSH
            mvbase = k * FLUSH

            def mv(i, _):
                stage[pl.ds(i * L, L)] = stage[pl.ds(mvbase + i * L, L)]
                return 0
            lax.fori_loop(0, (rem + L - 1) // L, mv, 0)
            return (v - k * FLUSH, o2)

        return lax.cond(vcs >= FLUSH, do_flush, lambda a: a, (vc, off))

    def chunk_pair(p, carry):
        for b in range(2):
            carry = chunk_one(p * 2 + b, b, carry)
        return carry

    vc, off = lax.fori_loop(0, E // SCAN_CH // 2, chunk_pair,
                            (jnp.zeros((L,), jnp.int32), jnp.int32(0)))
    fo = pl.multiple_of(wid * ECAP + off, 8)
    pltpu.sync_copy(stage.at[pl.ds(0, FLUSH)], pk_hbm.at[pl.ds(fo, FLUSH)])
    ct = off + jnp.max(vc)
    cbuf[pl.ds(0, L)] = jnp.zeros((L,), jnp.int32) + ct
    pltpu.sync_copy(cbuf.at[pl.ds(0, L)],
                    cnt_hbm.at[pl.ds(pl.multiple_of(wid * L, 8), L)])

    _gather_max_fold(x_hbm, pk_hbm, agg_hbm, pkv, idxv, ldv, rows, agg1d,
                     [sem0, sem1, sem2, sem3], wid, ct)


def _sc_layer2_body(h_hbm, pk_hbm, cnt_hbm,
                    agg_hbm,
                    cntv, pkv, idxv, ldv, rows, agg1d,
                    sem0, sem1, sem2, sem3):
    c = lax.axis_index("c")
    s = lax.axis_index("s")
    wid = c * NS + s
    pltpu.sync_copy(cnt_hbm, cntv)
    ct = jnp.max(cntv[pl.ds(wid * L, L)])
    _gather_max_fold(h_hbm, pk_hbm, agg_hbm, pkv, idxv, ldv, rows, agg1d,
                     [sem0, sem1, sem2, sem3], wid, ct)


def _pair(shape, dtype):
    return [pltpu.VMEM(shape, dtype), pltpu.VMEM(shape, dtype)]


def _sc_layer1(x, src, dst):
    f = pl.kernel(
        _sc_layer1_body,
        out_type=[
            jax.ShapeDtypeStruct((NPAD * D,), jnp.float32),
            jax.ShapeDtypeStruct((NW * ECAP,), jnp.int32),
            jax.ShapeDtypeStruct((NW * L,), jnp.int32),
        ],
        mesh=_mesh(),
        compiler_params=pltpu.CompilerParams(needs_layout_passes=False),
        scratch_types=[
            _pair((SCAN_CH,), jnp.int32),
            _pair((SCAN_CH,), jnp.int32),
            pltpu.VMEM((STAGE,), jnp.int32),
            _pair((GB,), jnp.int32),
            _pair((GB,), jnp.int32),
            _pair((GB,), jnp.int32),
            _pair((GB, D), jnp.float32),
            pltpu.VMEM(((RPW + 1) * D,), jnp.float32),
            pltpu.VMEM((L,), jnp.int32),
            pltpu.SemaphoreType.DMA,
            pltpu.SemaphoreType.DMA,
            pltpu.SemaphoreType.DMA,
            pltpu.SemaphoreType.DMA,
            pltpu.SemaphoreType.DMA,
        ],
    )
    return f(x, src, dst)


def _sc_layer2(h, pk, cnt):
    f = pl.kernel(
        _sc_layer2_body,
        out_type=jax.ShapeDtypeStruct((NPAD * D,), jnp.float32),
        mesh=_mesh(),
        compiler_params=pltpu.CompilerParams(needs_layout_passes=False),
        scratch_types=[
            pltpu.VMEM((NW * L,), jnp.int32),
            _pair((GB,), jnp.int32),
            _pair((GB,), jnp.int32),
            _pair((GB,), jnp.int32),
            _pair((GB, D), jnp.float32),
            pltpu.VMEM(((RPW + 1) * D,), jnp.float32),
            pltpu.SemaphoreType.DMA,
            pltpu.SemaphoreType.DMA,
            pltpu.SemaphoreType.DMA,
            pltpu.SemaphoreType.DMA,
        ],
    )
    return f(h, pk, cnt)


def _lin_body(relu, agg_ref, h_ref, wl_ref, wr_ref, b_ref, o_ref):
    a = agg_ref[...]
    a = jnp.where(a == -jnp.inf, 0.0, a)
    out = lax.dot_general(a, wl_ref[...], (((1,), (1,)), ((), ())),
                          preferred_element_type=jnp.float32)
    out = out + lax.dot_general(h_ref[...], wr_ref[...], (((1,), (1,)), ((), ())),
                                preferred_element_type=jnp.float32)
    out = out + b_ref[...]
    if relu:
        out = jnp.maximum(out, 0.0)
    o_ref[...] = out


def _linear(agg, h, W_l, b_l, W_r, relu):
    BM = 1000
    return pl.pallas_call(
        functools.partial(_lin_body, relu),
        grid=(N // BM,),
        in_specs=[
            pl.BlockSpec((BM, D), lambda i: (i, 0)),
            pl.BlockSpec((BM, D), lambda i: (i, 0)),
            pl.BlockSpec((D, D), lambda i: (0, 0)),
            pl.BlockSpec((D, D), lambda i: (0, 0)),
            pl.BlockSpec((1, D), lambda i: (0, 0)),
        ],
        out_specs=pl.BlockSpec((BM, D), lambda i: (i, 0)),
        out_shape=jax.ShapeDtypeStruct((N, D), jnp.float32),
    )(agg, h, W_l, W_r, b_l)


def kernel(x, edge_index, W1_l, b1_l, W1_r, W2_l, b2_l, W2_r):
    src = edge_index[0].astype(jnp.int32)
    dst = edge_index[1].astype(jnp.int32)
    agg1f, pk, cnt = _sc_layer1(x, src, dst)
    agg1 = agg1f.reshape(NPAD, D)[:N]
    h1 = _linear(agg1, x, W1_l, b1_l.reshape(1, D), W1_r, relu=True)
    agg2f = _sc_layer2(h1, pk, cnt)
    agg2 = agg2f.reshape(NPAD, D)[:N]
    return _linear(agg2, h1, W2_l, b2_l.reshape(1, D), W2_r, relu=False)


# speedup vs baseline: 3.3198x; 1.3154x over previous
import functools

import jax
import jax.numpy as jnp
from jax import lax
from jax.experimental import pallas as pl
from jax.experimental.pallas import tpu as pltpu
from jax.experimental.pallas import tpu_sc as plsc

N = 10000
E = 320000
D = 128
NC, NS, L = 2, 16, 16
NW = NC * NS
RPW = 313
NPAD = NW * RPW
SCAN_CH = 3200
FLUSH = 2048
STAGE = 4096 + 2048
ECAP = E + FLUSH
GB = 128
QD = D // L
SHIFT = 512

_mesh = lambda: plsc.VectorSubcoreMesh(core_axis_name="c", subcore_axis_name="s")


def _gather_max_fold(h_hbm, pk_hbm, agg_hbm, pkv, idxv, ldv, rows, agg1d,
                     sems, wid, ct):
    iota = lax.iota(jnp.int32, L)
    neg = jnp.full((L,), -jnp.inf, dtype=jnp.float32)

    def init_body(i, _):
        for q in range(16):
            agg1d[pl.ds(i * 256 + q * L, L)] = neg
        return 0
    lax.fori_loop(0, (RPW + 1) * D // 256, init_body, 0)

    nch = (ct + GB - 1) // GB

    def idx_start(g, b):
        base = pl.multiple_of(wid * ECAP + g * GB, 8)
        pltpu.async_copy(pk_hbm.at[pl.ds(base, GB)], pkv[b], sems[2 + b])

    def idx_wait_clean(g, b):
        pltpu.make_async_copy(pk_hbm.at[pl.ds(0, GB)], pkv[b],
                              sems[2 + b]).wait()
        for q in range(GB // L):
            w = pkv[b][pl.ds(q * L, L)]
            m = (g * GB + q * L + iota) < ct
            idxv[b][pl.ds(q * L, L)] = jnp.where(m, w // SHIFT, 0)
            ldv[b][pl.ds(q * L, L)] = jnp.where(m, w & (SHIFT - 1), RPW)

    def row_start(b):
        pltpu.async_copy(h_hbm.at[idxv[b]], rows[b], sems[b])

    def row_wait(b):
        pltpu.make_async_copy(h_hbm.at[pl.ds(0, GB)], rows[b], sems[b]).wait()

    def fold_chunk(b):
        def fold(p, _):
            esp0 = jnp.zeros((L,), jnp.int32) + p * 2
            esp1 = esp0 + 1
            lds0 = plsc.load_gather(ldv[b], [esp0])
            lds1 = plsc.load_gather(ldv[b], [esp1])
            same = lds0 == lds1
            ab0 = lds0 * D
            ab1 = lds1 * D
            for q in range(QD):
                col = q * L + iota
                r0 = plsc.load_gather(rows[b], [esp0, col])
                r1 = plsc.load_gather(rows[b], [esp1, col])
                comb = jnp.maximum(r0, r1)
                v0 = jnp.where(same, comb, r0)
                v1 = jnp.where(same, comb, r1)
                a0 = ab0 + col
                a1 = ab1 + col
                c0 = plsc.load_gather(agg1d, [a0])
                c1 = plsc.load_gather(agg1d, [a1])
                plsc.store_scatter(agg1d, [a0], jnp.maximum(c0, v0))
                plsc.store_scatter(agg1d, [a1], jnp.maximum(c1, v1))
            return 0
        lax.fori_loop(0, GB // 2, fold, 0)

    @pl.when(nch > 0)
    def _():
        idx_start(0, 0)
        idx_wait_clean(0, 0)
        row_start(0)

    @pl.when(nch > 1)
    def _():
        idx_start(1, 1)

    def pair(p, _):
        for b in range(2):
            g = p * 2 + b

            @pl.when(g < nch)
            def _():
                row_wait(b)

                @pl.when(g + 1 < nch)
                def _():
                    idx_wait_clean(g + 1, 1 - b)
                    row_start(1 - b)

                @pl.when(g + 2 < nch)
                def _():
                    idx_start(g + 2, b)

                fold_chunk(b)
        return 0
    lax.fori_loop(0, (nch + 1) // 2, pair, 0)

    pltpu.sync_copy(agg1d.at[pl.ds(0, RPW * D)],
                    agg_hbm.at[pl.ds(pl.multiple_of(wid * RPW * D, 8), RPW * D)])


def _sc_layer1_body(x_hbm, src_hbm, dst_hbm,
                    agg_hbm, pk_hbm, cnt_hbm,
                    dstv, srcv, stage, pkv, idxv, ldv, rows, agg1d,
                    cbuf, sem0, sem1, sem2, sem3, semd):
    c = lax.axis_index("c")
    s = lax.axis_index("s")
    wid = c * NS + s
    lo = wid * RPW
    hi = jnp.minimum(lo + RPW, N)
    iota = lax.iota(jnp.int32, L)

    def scan_start(ci, b):
        base = pl.multiple_of(ci * SCAN_CH, 8)
        pltpu.async_copy(dst_hbm.at[pl.ds(base, SCAN_CH)], dstv[b], semd)
        pltpu.async_copy(src_hbm.at[pl.ds(base, SCAN_CH)], srcv[b], semd)

    def scan_wait(b):
        pltpu.make_async_copy(dst_hbm.at[pl.ds(0, SCAN_CH)], dstv[b], semd).wait()
        pltpu.make_async_copy(src_hbm.at[pl.ds(0, SCAN_CH)], srcv[b], semd).wait()

    scan_start(0, 0)

    def chunk_one(ci, b, carry):
        vc, off = carry
        scan_wait(b)

        @pl.when(ci + 1 < E // SCAN_CH)
        def _():
            scan_start(ci + 1, 1 - b)

        def grp2(q, vcv):
            out = vcv
            for u in range(2):
                g = q * 2 + u
                d = dstv[b][pl.ds(g * L, L)]
                sv = srcv[b][pl.ds(g * L, L)]
                m = (d >= lo) & (d < hi)
                csum = jnp.cumsum(m.astype(jnp.int32))
                pos = out + csum - 1
                plsc.store_scatter(stage, [pos], sv * SHIFT + (d - lo), mask=m)
                out = out + plsc.all_reduce_population_count(m)
            return out
        vc = lax.fori_loop(0, SCAN_CH // L // 2, grp2, vc)
        vcs = jnp.max(vc)

        def do_flush(args):
            v, o = args
            k = vcs // FLUSH

            def fl(j, oo):
                so = pl.multiple_of(j * FLUSH, 8)
                fo = pl.multiple_of(wid * ECAP + oo, 8)
                pltpu.sync_copy(stage.at[pl.ds(so, FLUSH)],
                                pk_hbm.at[pl.ds(fo, FLUSH)])
                return oo + FLUSH
            o2 = lax.fori_loop(0, k, fl, o)
            rem = vcs - k * FLUSH
            mvbase = k * FLUSH

            def mv(i, _):
                stage[pl.ds(i * L, L)] = stage[pl.ds(mvbase + i * L, L)]
                return 0
            lax.fori_loop(0, (rem + L - 1) // L, mv, 0)
            return (v - k * FLUSH, o2)

        return lax.cond(vcs >= FLUSH, do_flush, lambda a: a, (vc, off))

    def chunk_pair(p, carry):
        for b in range(2):
            carry = chunk_one(p * 2 + b, b, carry)
        return carry

    vc, off = lax.fori_loop(0, E // SCAN_CH // 2, chunk_pair,
                            (jnp.zeros((L,), jnp.int32), jnp.int32(0)))
    fo = pl.multiple_of(wid * ECAP + off, 8)
    pltpu.sync_copy(stage.at[pl.ds(0, FLUSH)], pk_hbm.at[pl.ds(fo, FLUSH)])
    ct = off + jnp.max(vc)
    cbuf[pl.ds(0, L)] = jnp.zeros((L,), jnp.int32) + ct
    pltpu.sync_copy(cbuf.at[pl.ds(0, L)],
                    cnt_hbm.at[pl.ds(pl.multiple_of(wid * L, 8), L)])

    _gather_max_fold(x_hbm, pk_hbm, agg_hbm, pkv, idxv, ldv, rows, agg1d,
                     [sem0, sem1, sem2, sem3], wid, ct)


def _sc_layer2_body(h_hbm, pk_hbm, cnt_hbm,
                    agg_hbm,
                    cntv, pkv, idxv, ldv, rows, agg1d,
                    sem0, sem1, sem2, sem3):
    c = lax.axis_index("c")
    s = lax.axis_index("s")
    wid = c * NS + s
    pltpu.sync_copy(cnt_hbm, cntv)
    ct = jnp.max(cntv[pl.ds(wid * L, L)])
    _gather_max_fold(h_hbm, pk_hbm, agg_hbm, pkv, idxv, ldv, rows, agg1d,
                     [sem0, sem1, sem2, sem3], wid, ct)


def _pair(shape, dtype):
    return [pltpu.VMEM(shape, dtype), pltpu.VMEM(shape, dtype)]


def _sc_layer1(x, src, dst):
    f = pl.kernel(
        _sc_layer1_body,
        out_type=[
            jax.ShapeDtypeStruct((NPAD * D,), jnp.float32),
            jax.ShapeDtypeStruct((NW * ECAP,), jnp.int32),
            jax.ShapeDtypeStruct((NW * L,), jnp.int32),
        ],
        mesh=_mesh(),
        compiler_params=pltpu.CompilerParams(needs_layout_passes=False),
        scratch_types=[
            _pair((SCAN_CH,), jnp.int32),
            _pair((SCAN_CH,), jnp.int32),
            pltpu.VMEM((STAGE,), jnp.int32),
            _pair((GB,), jnp.int32),
            _pair((GB,), jnp.int32),
            _pair((GB,), jnp.int32),
            _pair((GB, D), jnp.float32),
            pltpu.VMEM(((RPW + 1) * D,), jnp.float32),
            pltpu.VMEM((L,), jnp.int32),
            pltpu.SemaphoreType.DMA,
            pltpu.SemaphoreType.DMA,
            pltpu.SemaphoreType.DMA,
            pltpu.SemaphoreType.DMA,
            pltpu.SemaphoreType.DMA,
        ],
    )
    return f(x, src, dst)


def _sc_layer2(h, pk, cnt):
    f = pl.kernel(
        _sc_layer2_body,
        out_type=jax.ShapeDtypeStruct((NPAD * D,), jnp.float32),
        mesh=_mesh(),
        compiler_params=pltpu.CompilerParams(needs_layout_passes=False),
        scratch_types=[
            pltpu.VMEM((NW * L,), jnp.int32),
            _pair((GB,), jnp.int32),
            _pair((GB,), jnp.int32),
            _pair((GB,), jnp.int32),
            _pair((GB, D), jnp.float32),
            pltpu.VMEM(((RPW + 1) * D,), jnp.float32),
            pltpu.SemaphoreType.DMA,
            pltpu.SemaphoreType.DMA,
            pltpu.SemaphoreType.DMA,
            pltpu.SemaphoreType.DMA,
        ],
    )
    return f(h, pk, cnt)


def _lin_body(relu, agg_ref, h_ref, wl_ref, wr_ref, b_ref, o_ref):
    a = agg_ref[...]
    a = jnp.where(a == -jnp.inf, 0.0, a)
    out = lax.dot_general(a, wl_ref[...], (((1,), (1,)), ((), ())),
                          preferred_element_type=jnp.float32)
    out = out + lax.dot_general(h_ref[...], wr_ref[...], (((1,), (1,)), ((), ())),
                                preferred_element_type=jnp.float32)
    out = out + b_ref[...]
    if relu:
        out = jnp.maximum(out, 0.0)
    o_ref[...] = out


def _linear(agg, h, W_l, b_l, W_r, relu):
    BM = 1000
    return pl.pallas_call(
        functools.partial(_lin_body, relu),
        grid=(N // BM,),
        in_specs=[
            pl.BlockSpec((BM, D), lambda i: (i, 0)),
            pl.BlockSpec((BM, D), lambda i: (i, 0)),
            pl.BlockSpec((D, D), lambda i: (0, 0)),
            pl.BlockSpec((D, D), lambda i: (0, 0)),
            pl.BlockSpec((1, D), lambda i: (0, 0)),
        ],
        out_specs=pl.BlockSpec((BM, D), lambda i: (i, 0)),
        out_shape=jax.ShapeDtypeStruct((N, D), jnp.float32),
    )(agg, h, W_l, W_r, b_l)


def kernel(x, edge_index, W1_l, b1_l, W1_r, W2_l, b2_l, W2_r):
    src = edge_index[0].astype(jnp.int32)
    dst = edge_index[1].astype(jnp.int32)
    agg1f, pk, cnt = _sc_layer1(x, src, dst)
    agg1 = agg1f.reshape(NPAD, D)[:N]
    h1 = _linear(agg1, x, W1_l, b1_l.reshape(1, D), W1_r, relu=True)
    agg2f = _sc_layer2(h1, pk, cnt)
    agg2 = agg2f.reshape(NPAD, D)[:N]
    return _linear(agg2, h1, W2_l, b2_l.reshape(1, D), W2_r, relu=False)


# speedup vs baseline: 3.4312x; 1.0336x over previous
import functools

import jax
import jax.numpy as jnp
from jax import lax
from jax.experimental import pallas as pl
from jax.experimental.pallas import tpu as pltpu
from jax.experimental.pallas import tpu_sc as plsc

N = 10000
E = 320000
D = 128
NC, NS, L = 2, 16, 16
NW = NC * NS
RPW = 313
NPAD = NW * RPW
SCAN_CH = 3200
FLUSH = 2048
STAGE = 4096 + 2048
ECAP = E + FLUSH
GB = 128
QD = D // L
SHIFT = 512

_mesh = lambda: plsc.VectorSubcoreMesh(core_axis_name="c", subcore_axis_name="s")


def _gather_max_fold(h_hbm, pk_hbm, agg_hbm, pkv, idxv, ldv, rows, agg1d,
                     sems, wid, ct):
    iota = lax.iota(jnp.int32, L)
    neg = jnp.full((L,), -jnp.inf, dtype=jnp.float32)

    def init_body(i, _):
        for q in range(16):
            agg1d[pl.ds(i * 256 + q * L, L)] = neg
        return 0
    lax.fori_loop(0, (RPW + 1) * D // 256, init_body, 0)

    nch = (ct + GB - 1) // GB

    def idx_start(g, b):
        base = pl.multiple_of(wid * ECAP + g * GB, 8)
        pltpu.async_copy(pk_hbm.at[pl.ds(base, GB)], pkv[b], sems[2 + b])

    def idx_wait_clean(g, b):
        pltpu.make_async_copy(pk_hbm.at[pl.ds(0, GB)], pkv[b],
                              sems[2 + b]).wait()
        for q in range(GB // L):
            w = pkv[b][pl.ds(q * L, L)]
            m = (g * GB + q * L + iota) < ct
            idxv[b][pl.ds(q * L, L)] = jnp.where(m, w // SHIFT, 0)
            ldv[b][pl.ds(q * L, L)] = jnp.where(m, w & (SHIFT - 1), RPW)

    def row_start(b):
        pltpu.async_copy(h_hbm.at[idxv[b]], rows[b], sems[b])

    def row_wait(b):
        pltpu.make_async_copy(h_hbm.at[pl.ds(0, GB)], rows[b], sems[b]).wait()

    def fold_chunk(b):
        def pair_block(e0):
            esp0 = jnp.zeros((L,), jnp.int32) + e0
            esp1 = esp0 + 1
            lds0 = plsc.load_gather(ldv[b], [esp0])
            lds1 = plsc.load_gather(ldv[b], [esp1])
            same = lds0 == lds1
            ab0 = lds0 * D
            ab1 = lds1 * D
            for q in range(QD):
                col = q * L + iota
                r0 = plsc.load_gather(rows[b], [esp0, col])
                r1 = plsc.load_gather(rows[b], [esp1, col])
                comb = jnp.maximum(r0, r1)
                v0 = jnp.where(same, comb, r0)
                v1 = jnp.where(same, comb, r1)
                a0 = ab0 + col
                a1 = ab1 + col
                c0 = plsc.load_gather(agg1d, [a0])
                c1 = plsc.load_gather(agg1d, [a1])
                plsc.store_scatter(agg1d, [a0], jnp.maximum(c0, v0))
                plsc.store_scatter(agg1d, [a1], jnp.maximum(c1, v1))

        def fold(p, _):
            pair_block(p * 4)
            pair_block(p * 4 + 2)
            return 0
        lax.fori_loop(0, GB // 4, fold, 0)

    @pl.when(nch > 0)
    def _():
        idx_start(0, 0)
        idx_wait_clean(0, 0)
        row_start(0)

    @pl.when(nch > 1)
    def _():
        idx_start(1, 1)

    def pair(p, _):
        for b in range(2):
            g = p * 2 + b

            @pl.when(g < nch)
            def _():
                row_wait(b)

                @pl.when(g + 1 < nch)
                def _():
                    idx_wait_clean(g + 1, 1 - b)
                    row_start(1 - b)

                @pl.when(g + 2 < nch)
                def _():
                    idx_start(g + 2, b)

                fold_chunk(b)
        return 0
    lax.fori_loop(0, (nch + 1) // 2, pair, 0)

    pltpu.sync_copy(agg1d.at[pl.ds(0, RPW * D)],
                    agg_hbm.at[pl.ds(pl.multiple_of(wid * RPW * D, 8), RPW * D)])


def _sc_layer1_body(x_hbm, src_hbm, dst_hbm,
                    agg_hbm, pk_hbm, cnt_hbm,
                    dstv, srcv, stage, pkv, idxv, ldv, rows, agg1d,
                    cbuf, sem0, sem1, sem2, sem3, semd):
    c = lax.axis_index("c")
    s = lax.axis_index("s")
    wid = c * NS + s
    lo = wid * RPW
    hi = jnp.minimum(lo + RPW, N)
    iota = lax.iota(jnp.int32, L)

    def scan_start(ci, b):
        base = pl.multiple_of(ci * SCAN_CH, 8)
        pltpu.async_copy(dst_hbm.at[pl.ds(base, SCAN_CH)], dstv[b], semd)
        pltpu.async_copy(src_hbm.at[pl.ds(base, SCAN_CH)], srcv[b], semd)

    def scan_wait(b):
        pltpu.make_async_copy(dst_hbm.at[pl.ds(0, SCAN_CH)], dstv[b], semd).wait()
        pltpu.make_async_copy(src_hbm.at[pl.ds(0, SCAN_CH)], srcv[b], semd).wait()

    scan_start(0, 0)

    def chunk_one(ci, b, carry):
        vc, off = carry
        scan_wait(b)

        @pl.when(ci + 1 < E // SCAN_CH)
        def _():
            scan_start(ci + 1, 1 - b)

        def grp2(q, vcv):
            out = vcv
            for u in range(2):
                g = q * 2 + u
                d = dstv[b][pl.ds(g * L, L)]
                sv = srcv[b][pl.ds(g * L, L)]
                m = (d >= lo) & (d < hi)
                csum = jnp.cumsum(m.astype(jnp.int32))
                pos = out + csum - 1
                plsc.store_scatter(stage, [pos], sv * SHIFT + (d - lo), mask=m)
                out = out + plsc.all_reduce_population_count(m)
            return out
        vc = lax.fori_loop(0, SCAN_CH // L // 2, grp2, vc)
        vcs = jnp.max(vc)

        def do_flush(args):
            v, o = args
            k = vcs // FLUSH

            def fl(j, oo):
                so = pl.multiple_of(j * FLUSH, 8)
                fo = pl.multiple_of(wid * ECAP + oo, 8)
                pltpu.sync_copy(stage.at[pl.ds(so, FLUSH)],
                                pk_hbm.at[pl.ds(fo, FLUSH)])
                return oo + FLUSH
            o2 = lax.fori_loop(0, k, fl, o)
            rem = vcs - k * FLUSH
            mvbase = k * FLUSH

            def mv(i, _):
                stage[pl.ds(i * L, L)] = stage[pl.ds(mvbase + i * L, L)]
                return 0
            lax.fori_loop(0, (rem + L - 1) // L, mv, 0)
            return (v - k * FLUSH, o2)

        return lax.cond(vcs >= FLUSH, do_flush, lambda a: a, (vc, off))

    def chunk_pair(p, carry):
        for b in range(2):
            carry = chunk_one(p * 2 + b, b, carry)
        return carry

    vc, off = lax.fori_loop(0, E // SCAN_CH // 2, chunk_pair,
                            (jnp.zeros((L,), jnp.int32), jnp.int32(0)))
    fo = pl.multiple_of(wid * ECAP + off, 8)
    pltpu.sync_copy(stage.at[pl.ds(0, FLUSH)], pk_hbm.at[pl.ds(fo, FLUSH)])
    ct = off + jnp.max(vc)
    cbuf[pl.ds(0, L)] = jnp.zeros((L,), jnp.int32) + ct
    pltpu.sync_copy(cbuf.at[pl.ds(0, L)],
                    cnt_hbm.at[pl.ds(pl.multiple_of(wid * L, 8), L)])

    _gather_max_fold(x_hbm, pk_hbm, agg_hbm, pkv, idxv, ldv, rows, agg1d,
                     [sem0, sem1, sem2, sem3], wid, ct)


def _sc_layer2_body(h_hbm, pk_hbm, cnt_hbm,
                    agg_hbm,
                    cntv, pkv, idxv, ldv, rows, agg1d,
                    sem0, sem1, sem2, sem3):
    c = lax.axis_index("c")
    s = lax.axis_index("s")
    wid = c * NS + s
    pltpu.sync_copy(cnt_hbm, cntv)
    ct = jnp.max(cntv[pl.ds(wid * L, L)])
    _gather_max_fold(h_hbm, pk_hbm, agg_hbm, pkv, idxv, ldv, rows, agg1d,
                     [sem0, sem1, sem2, sem3], wid, ct)


def _pair(shape, dtype):
    return [pltpu.VMEM(shape, dtype), pltpu.VMEM(shape, dtype)]


def _sc_layer1(x, src, dst):
    f = pl.kernel(
        _sc_layer1_body,
        out_type=[
            jax.ShapeDtypeStruct((NPAD * D,), jnp.float32),
            jax.ShapeDtypeStruct((NW * ECAP,), jnp.int32),
            jax.ShapeDtypeStruct((NW * L,), jnp.int32),
        ],
        mesh=_mesh(),
        compiler_params=pltpu.CompilerParams(needs_layout_passes=False, disable_bounds_checks=True),
        scratch_types=[
            _pair((SCAN_CH,), jnp.int32),
            _pair((SCAN_CH,), jnp.int32),
            pltpu.VMEM((STAGE,), jnp.int32),
            _pair((GB,), jnp.int32),
            _pair((GB,), jnp.int32),
            _pair((GB,), jnp.int32),
            _pair((GB, D), jnp.float32),
            pltpu.VMEM(((RPW + 1) * D,), jnp.float32),
            pltpu.VMEM((L,), jnp.int32),
            pltpu.SemaphoreType.DMA,
            pltpu.SemaphoreType.DMA,
            pltpu.SemaphoreType.DMA,
            pltpu.SemaphoreType.DMA,
            pltpu.SemaphoreType.DMA,
        ],
    )
    return f(x, src, dst)


def _sc_layer2(h, pk, cnt):
    f = pl.kernel(
        _sc_layer2_body,
        out_type=jax.ShapeDtypeStruct((NPAD * D,), jnp.float32),
        mesh=_mesh(),
        compiler_params=pltpu.CompilerParams(needs_layout_passes=False, disable_bounds_checks=True),
        scratch_types=[
            pltpu.VMEM((NW * L,), jnp.int32),
            _pair((GB,), jnp.int32),
            _pair((GB,), jnp.int32),
            _pair((GB,), jnp.int32),
            _pair((GB, D), jnp.float32),
            pltpu.VMEM(((RPW + 1) * D,), jnp.float32),
            pltpu.SemaphoreType.DMA,
            pltpu.SemaphoreType.DMA,
            pltpu.SemaphoreType.DMA,
            pltpu.SemaphoreType.DMA,
        ],
    )
    return f(h, pk, cnt)


def _lin_body(relu, agg_ref, h_ref, wl_ref, wr_ref, b_ref, o_ref):
    a = agg_ref[...]
    a = jnp.where(a == -jnp.inf, 0.0, a)
    out = lax.dot_general(a, wl_ref[...], (((1,), (1,)), ((), ())),
                          preferred_element_type=jnp.float32)
    out = out + lax.dot_general(h_ref[...], wr_ref[...], (((1,), (1,)), ((), ())),
                                preferred_element_type=jnp.float32)
    out = out + b_ref[...]
    if relu:
        out = jnp.maximum(out, 0.0)
    o_ref[...] = out


def _linear(agg, h, W_l, b_l, W_r, relu):
    BM = 1000
    return pl.pallas_call(
        functools.partial(_lin_body, relu),
        grid=(N // BM,),
        in_specs=[
            pl.BlockSpec((BM, D), lambda i: (i, 0)),
            pl.BlockSpec((BM, D), lambda i: (i, 0)),
            pl.BlockSpec((D, D), lambda i: (0, 0)),
            pl.BlockSpec((D, D), lambda i: (0, 0)),
            pl.BlockSpec((1, D), lambda i: (0, 0)),
        ],
        out_specs=pl.BlockSpec((BM, D), lambda i: (i, 0)),
        out_shape=jax.ShapeDtypeStruct((N, D), jnp.float32),
    )(agg, h, W_l, W_r, b_l)


def kernel(x, edge_index, W1_l, b1_l, W1_r, W2_l, b2_l, W2_r):
    src = edge_index[0].astype(jnp.int32)
    dst = edge_index[1].astype(jnp.int32)
    agg1f, pk, cnt = _sc_layer1(x, src, dst)
    agg1 = agg1f.reshape(NPAD, D)[:N]
    h1 = _linear(agg1, x, W1_l, b1_l.reshape(1, D), W1_r, relu=True)
    agg2f = _sc_layer2(h1, pk, cnt)
    agg2 = agg2f.reshape(NPAD, D)[:N]
    return _linear(agg2, h1, W2_l, b2_l.reshape(1, D), W2_r, relu=False)


# speedup vs baseline: 4.4882x; 1.3080x over previous
import functools

import jax
import jax.numpy as jnp
from jax import lax
from jax.experimental import pallas as pl
from jax.experimental.pallas import tpu as pltpu
from jax.experimental.pallas import tpu_sc as plsc

N = 10000
E = 320000
D = 128
NC, NS, L = 2, 16, 16
NW = NC * NS
RPW = 313
NPAD = NW * RPW
SCAN_CH = 3200
FLUSH = 2048
STAGE = 4096 + 2048
ECAP = E + FLUSH
GB = 128
QD = D // L
SHIFT = 512

_mesh = lambda: plsc.VectorSubcoreMesh(core_axis_name="c", subcore_axis_name="s")


def _gather_max_fold(h_hbm, pk_hbm, agg_hbm, pkv, idxv, ldv, rows, agg1d,
                     sems, wid, ct):
    iota = lax.iota(jnp.int32, L)
    neg = jnp.full((L,), -jnp.inf, dtype=jnp.float32)

    def init_body(i, _):
        for q in range(16):
            agg1d[pl.ds(i * 256 + q * L, L)] = neg
        return 0
    lax.fori_loop(0, (RPW + 1) * D // 256, init_body, 0)

    nch = (ct + GB - 1) // GB

    def idx_start(g, b):
        base = pl.multiple_of(wid * ECAP + g * GB, 8)
        pltpu.async_copy(pk_hbm.at[pl.ds(base, GB)], pkv[b], sems[2 + b])

    def idx_wait_clean(g, b):
        pltpu.make_async_copy(pk_hbm.at[pl.ds(0, GB)], pkv[b],
                              sems[2 + b]).wait()
        for q in range(GB // L):
            w = pkv[b][pl.ds(q * L, L)]
            m = (g * GB + q * L + iota) < ct
            idxv[b][pl.ds(q * L, L)] = jnp.where(m, w // SHIFT, 0)
            ldv[b][pl.ds(q * L, L)] = jnp.where(m, w & (SHIFT - 1), RPW)

    def row_start(b):
        pltpu.async_copy(h_hbm.at[idxv[b]], rows[b], sems[b])

    def row_wait(b):
        pltpu.make_async_copy(h_hbm.at[pl.ds(0, GB)], rows[b], sems[b]).wait()

    def fold_chunk(b):
        def pair_block(e0):
            esp0 = jnp.zeros((L,), jnp.int32) + e0
            esp1 = esp0 + 1
            lds0 = plsc.load_gather(ldv[b], [esp0])
            lds1 = plsc.load_gather(ldv[b], [esp1])
            same = lds0 == lds1
            ab0 = lds0 * D
            ab1 = lds1 * D
            cols = [q * L + iota for q in range(QD)]
            a0s = [ab0 + c for c in cols]
            a1s = [ab1 + c for c in cols]
            r0s = [plsc.load_gather(rows[b], [esp0, c]) for c in cols]
            r1s = [plsc.load_gather(rows[b], [esp1, c]) for c in cols]
            c0s = [plsc.load_gather(agg1d, [a]) for a in a0s]
            c1s = [plsc.load_gather(agg1d, [a]) for a in a1s]
            for q in range(QD):
                comb = jnp.maximum(r0s[q], r1s[q])
                v0 = jnp.where(same, comb, r0s[q])
                v1 = jnp.where(same, comb, r1s[q])
                plsc.store_scatter(agg1d, [a0s[q]], jnp.maximum(c0s[q], v0))
                plsc.store_scatter(agg1d, [a1s[q]], jnp.maximum(c1s[q], v1))

        def fold(p, _):
            pair_block(p * 4)
            pair_block(p * 4 + 2)
            return 0
        lax.fori_loop(0, GB // 4, fold, 0)

    @pl.when(nch > 0)
    def _():
        idx_start(0, 0)
        idx_wait_clean(0, 0)
        row_start(0)

    @pl.when(nch > 1)
    def _():
        idx_start(1, 1)

    def pair(p, _):
        for b in range(2):
            g = p * 2 + b

            @pl.when(g < nch)
            def _():
                row_wait(b)

                @pl.when(g + 1 < nch)
                def _():
                    idx_wait_clean(g + 1, 1 - b)
                    row_start(1 - b)

                @pl.when(g + 2 < nch)
                def _():
                    idx_start(g + 2, b)

                fold_chunk(b)
        return 0
    lax.fori_loop(0, (nch + 1) // 2, pair, 0)

    pltpu.sync_copy(agg1d.at[pl.ds(0, RPW * D)],
                    agg_hbm.at[pl.ds(pl.multiple_of(wid * RPW * D, 8), RPW * D)])


def _sc_layer1_body(x_hbm, src_hbm, dst_hbm,
                    agg_hbm, pk_hbm, cnt_hbm,
                    dstv, srcv, stage, pkv, idxv, ldv, rows, agg1d,
                    cbuf, sem0, sem1, sem2, sem3, semd):
    c = lax.axis_index("c")
    s = lax.axis_index("s")
    wid = c * NS + s
    lo = wid * RPW
    hi = jnp.minimum(lo + RPW, N)
    iota = lax.iota(jnp.int32, L)

    def scan_start(ci, b):
        base = pl.multiple_of(ci * SCAN_CH, 8)
        pltpu.async_copy(dst_hbm.at[pl.ds(base, SCAN_CH)], dstv[b], semd)
        pltpu.async_copy(src_hbm.at[pl.ds(base, SCAN_CH)], srcv[b], semd)

    def scan_wait(b):
        pltpu.make_async_copy(dst_hbm.at[pl.ds(0, SCAN_CH)], dstv[b], semd).wait()
        pltpu.make_async_copy(src_hbm.at[pl.ds(0, SCAN_CH)], srcv[b], semd).wait()

    scan_start(0, 0)

    def chunk_one(ci, b, carry):
        vc, off = carry
        scan_wait(b)

        @pl.when(ci + 1 < E // SCAN_CH)
        def _():
            scan_start(ci + 1, 1 - b)

        def grp2(q, vcv):
            out = vcv
            for u in range(2):
                g = q * 2 + u
                d = dstv[b][pl.ds(g * L, L)]
                sv = srcv[b][pl.ds(g * L, L)]
                m = (d >= lo) & (d < hi)
                csum = jnp.cumsum(m.astype(jnp.int32))
                pos = out + csum - 1
                plsc.store_scatter(stage, [pos], sv * SHIFT + (d - lo), mask=m)
                out = out + plsc.all_reduce_population_count(m)
            return out
        vc = lax.fori_loop(0, SCAN_CH // L // 2, grp2, vc)
        vcs = jnp.max(vc)

        def do_flush(args):
            v, o = args
            k = vcs // FLUSH

            def fl(j, oo):
                so = pl.multiple_of(j * FLUSH, 8)
                fo = pl.multiple_of(wid * ECAP + oo, 8)
                pltpu.sync_copy(stage.at[pl.ds(so, FLUSH)],
                                pk_hbm.at[pl.ds(fo, FLUSH)])
                return oo + FLUSH
            o2 = lax.fori_loop(0, k, fl, o)
            rem = vcs - k * FLUSH
            mvbase = k * FLUSH

            def mv(i, _):
                stage[pl.ds(i * L, L)] = stage[pl.ds(mvbase + i * L, L)]
                return 0
            lax.fori_loop(0, (rem + L - 1) // L, mv, 0)
            return (v - k * FLUSH, o2)

        return lax.cond(vcs >= FLUSH, do_flush, lambda a: a, (vc, off))

    def chunk_pair(p, carry):
        for b in range(2):
            carry = chunk_one(p * 2 + b, b, carry)
        return carry

    vc, off = lax.fori_loop(0, E // SCAN_CH // 2, chunk_pair,
                            (jnp.zeros((L,), jnp.int32), jnp.int32(0)))
    fo = pl.multiple_of(wid * ECAP + off, 8)
    pltpu.sync_copy(stage.at[pl.ds(0, FLUSH)], pk_hbm.at[pl.ds(fo, FLUSH)])
    ct = off + jnp.max(vc)
    cbuf[pl.ds(0, L)] = jnp.zeros((L,), jnp.int32) + ct
    pltpu.sync_copy(cbuf.at[pl.ds(0, L)],
                    cnt_hbm.at[pl.ds(pl.multiple_of(wid * L, 8), L)])

    _gather_max_fold(x_hbm, pk_hbm, agg_hbm, pkv, idxv, ldv, rows, agg1d,
                     [sem0, sem1, sem2, sem3], wid, ct)


def _sc_layer2_body(h_hbm, pk_hbm, cnt_hbm,
                    agg_hbm,
                    cntv, pkv, idxv, ldv, rows, agg1d,
                    sem0, sem1, sem2, sem3):
    c = lax.axis_index("c")
    s = lax.axis_index("s")
    wid = c * NS + s
    pltpu.sync_copy(cnt_hbm, cntv)
    ct = jnp.max(cntv[pl.ds(wid * L, L)])
    _gather_max_fold(h_hbm, pk_hbm, agg_hbm, pkv, idxv, ldv, rows, agg1d,
                     [sem0, sem1, sem2, sem3], wid, ct)


def _pair(shape, dtype):
    return [pltpu.VMEM(shape, dtype), pltpu.VMEM(shape, dtype)]


def _sc_layer1(x, src, dst):
    f = pl.kernel(
        _sc_layer1_body,
        out_type=[
            jax.ShapeDtypeStruct((NPAD * D,), jnp.float32),
            jax.ShapeDtypeStruct((NW * ECAP,), jnp.int32),
            jax.ShapeDtypeStruct((NW * L,), jnp.int32),
        ],
        mesh=_mesh(),
        compiler_params=pltpu.CompilerParams(needs_layout_passes=False, disable_bounds_checks=True),
        scratch_types=[
            _pair((SCAN_CH,), jnp.int32),
            _pair((SCAN_CH,), jnp.int32),
            pltpu.VMEM((STAGE,), jnp.int32),
            _pair((GB,), jnp.int32),
            _pair((GB,), jnp.int32),
            _pair((GB,), jnp.int32),
            _pair((GB, D), jnp.float32),
            pltpu.VMEM(((RPW + 1) * D,), jnp.float32),
            pltpu.VMEM((L,), jnp.int32),
            pltpu.SemaphoreType.DMA,
            pltpu.SemaphoreType.DMA,
            pltpu.SemaphoreType.DMA,
            pltpu.SemaphoreType.DMA,
            pltpu.SemaphoreType.DMA,
        ],
    )
    return f(x, src, dst)


def _sc_layer2(h, pk, cnt):
    f = pl.kernel(
        _sc_layer2_body,
        out_type=jax.ShapeDtypeStruct((NPAD * D,), jnp.float32),
        mesh=_mesh(),
        compiler_params=pltpu.CompilerParams(needs_layout_passes=False, disable_bounds_checks=True),
        scratch_types=[
            pltpu.VMEM((NW * L,), jnp.int32),
            _pair((GB,), jnp.int32),
            _pair((GB,), jnp.int32),
            _pair((GB,), jnp.int32),
            _pair((GB, D), jnp.float32),
            pltpu.VMEM(((RPW + 1) * D,), jnp.float32),
            pltpu.SemaphoreType.DMA,
            pltpu.SemaphoreType.DMA,
            pltpu.SemaphoreType.DMA,
            pltpu.SemaphoreType.DMA,
        ],
    )
    return f(h, pk, cnt)


def _lin_body(relu, agg_ref, h_ref, wl_ref, wr_ref, b_ref, o_ref):
    a = agg_ref[...]
    a = jnp.where(a == -jnp.inf, 0.0, a)
    out = lax.dot_general(a, wl_ref[...], (((1,), (1,)), ((), ())),
                          preferred_element_type=jnp.float32)
    out = out + lax.dot_general(h_ref[...], wr_ref[...], (((1,), (1,)), ((), ())),
                                preferred_element_type=jnp.float32)
    out = out + b_ref[...]
    if relu:
        out = jnp.maximum(out, 0.0)
    o_ref[...] = out


def _linear(agg, h, W_l, b_l, W_r, relu):
    BM = 1000
    return pl.pallas_call(
        functools.partial(_lin_body, relu),
        grid=(N // BM,),
        in_specs=[
            pl.BlockSpec((BM, D), lambda i: (i, 0)),
            pl.BlockSpec((BM, D), lambda i: (i, 0)),
            pl.BlockSpec((D, D), lambda i: (0, 0)),
            pl.BlockSpec((D, D), lambda i: (0, 0)),
            pl.BlockSpec((1, D), lambda i: (0, 0)),
        ],
        out_specs=pl.BlockSpec((BM, D), lambda i: (i, 0)),
        out_shape=jax.ShapeDtypeStruct((N, D), jnp.float32),
    )(agg, h, W_l, W_r, b_l)


def kernel(x, edge_index, W1_l, b1_l, W1_r, W2_l, b2_l, W2_r):
    src = edge_index[0].astype(jnp.int32)
    dst = edge_index[1].astype(jnp.int32)
    agg1f, pk, cnt = _sc_layer1(x, src, dst)
    agg1 = agg1f.reshape(NPAD, D)[:N]
    h1 = _linear(agg1, x, W1_l, b1_l.reshape(1, D), W1_r, relu=True)
    agg2f = _sc_layer2(h1, pk, cnt)
    agg2 = agg2f.reshape(NPAD, D)[:N]
    return _linear(agg2, h1, W2_l, b2_l.reshape(1, D), W2_r, relu=False)


# speedup vs baseline: 4.5580x; 1.0155x over previous
import functools

import jax
import jax.numpy as jnp
from jax import lax
from jax.experimental import pallas as pl
from jax.experimental.pallas import tpu as pltpu
from jax.experimental.pallas import tpu_sc as plsc

N = 10000
E = 320000
D = 128
NC, NS, L = 2, 16, 16
NW = NC * NS
RPW = 313
NPAD = NW * RPW
SCAN_CH = 3200
FLUSH = 2048
STAGE = 4096 + 2048
ECAP = E + FLUSH
GB = 128
QD = D // L
SHIFT = 512

_mesh = lambda: plsc.VectorSubcoreMesh(core_axis_name="c", subcore_axis_name="s")


def _gather_max_fold(h_hbm, pk_hbm, agg_hbm, pkv, idxv, ldv, rows, agg1d,
                     sems, wid, ct):
    iota = lax.iota(jnp.int32, L)
    neg = jnp.full((L,), -jnp.inf, dtype=jnp.float32)

    def init_body(i, _):
        for q in range(16):
            agg1d[pl.ds(i * 256 + q * L, L)] = neg
        return 0
    lax.fori_loop(0, (RPW + 1) * D // 256, init_body, 0)

    nch = (ct + GB - 1) // GB

    def idx_start(g, b):
        base = pl.multiple_of(wid * ECAP + g * GB, 8)
        pltpu.async_copy(pk_hbm.at[pl.ds(base, GB)], pkv[b], sems[2 + b])

    def idx_wait_clean(g, b):
        pltpu.make_async_copy(pk_hbm.at[pl.ds(0, GB)], pkv[b],
                              sems[2 + b]).wait()
        for q in range(GB // L):
            w = pkv[b][pl.ds(q * L, L)]
            m = (g * GB + q * L + iota) < ct
            idxv[b][pl.ds(q * L, L)] = jnp.where(m, w // SHIFT, 0)
            ldv[b][pl.ds(q * L, L)] = jnp.where(m, w & (SHIFT - 1), RPW)

    def row_start(b):
        pltpu.async_copy(h_hbm.at[idxv[b]], rows[b], sems[b])

    def row_wait(b):
        pltpu.make_async_copy(h_hbm.at[pl.ds(0, GB)], rows[b], sems[b]).wait()

    def fold_chunk(b):
        def pair_block(e0):
            esp0 = jnp.zeros((L,), jnp.int32) + e0
            esp1 = esp0 + 1
            lds0 = plsc.load_gather(ldv[b], [esp0])
            lds1 = plsc.load_gather(ldv[b], [esp1])
            same = lds0 == lds1
            ab0 = lds0 * D
            ab1 = lds1 * D
            cols = [q * L + iota for q in range(QD)]
            a0s = [ab0 + c for c in cols]
            a1s = [ab1 + c for c in cols]
            r0s = [plsc.load_gather(rows[b], [esp0, c]) for c in cols]
            r1s = [plsc.load_gather(rows[b], [esp1, c]) for c in cols]
            c0s = [plsc.load_gather(agg1d, [a]) for a in a0s]
            c1s = [plsc.load_gather(agg1d, [a]) for a in a1s]
            for q in range(QD):
                comb = jnp.maximum(r0s[q], r1s[q])
                v0 = jnp.where(same, comb, r0s[q])
                v1 = jnp.where(same, comb, r1s[q])
                plsc.store_scatter(agg1d, [a0s[q]], jnp.maximum(c0s[q], v0))
                plsc.store_scatter(agg1d, [a1s[q]], jnp.maximum(c1s[q], v1))

        def fold(p, _):
            pair_block(p * 4)
            pair_block(p * 4 + 2)
            return 0
        lax.fori_loop(0, GB // 4, fold, 0)

    @pl.when(nch > 0)
    def _():
        idx_start(0, 0)
        idx_wait_clean(0, 0)
        row_start(0)

    @pl.when(nch > 1)
    def _():
        idx_start(1, 1)

    def pair(p, _):
        for b in range(2):
            g = p * 2 + b

            @pl.when(g < nch)
            def _():
                row_wait(b)

                @pl.when(g + 1 < nch)
                def _():
                    idx_wait_clean(g + 1, 1 - b)
                    row_start(1 - b)

                @pl.when(g + 2 < nch)
                def _():
                    idx_start(g + 2, b)

                fold_chunk(b)
        return 0
    lax.fori_loop(0, (nch + 1) // 2, pair, 0)

    pltpu.sync_copy(agg1d.at[pl.ds(0, RPW * D)],
                    agg_hbm.at[pl.ds(pl.multiple_of(wid * RPW * D, 8), RPW * D)])


def _sc_layer1_body(x_hbm, src_hbm, dst_hbm,
                    agg_hbm, pk_hbm, cnt_hbm,
                    dstv, srcv, stage, pkv, idxv, ldv, rows, agg1d,
                    cbuf, sem0, sem1, sem2, sem3, semd):
    c = lax.axis_index("c")
    s = lax.axis_index("s")
    wid = c * NS + s
    lo = wid * RPW
    hi = jnp.minimum(lo + RPW, N)
    iota = lax.iota(jnp.int32, L)

    def scan_start(ci, b):
        base = pl.multiple_of(ci * SCAN_CH, 8)
        pltpu.async_copy(dst_hbm.at[pl.ds(base, SCAN_CH)], dstv[b], semd)
        pltpu.async_copy(src_hbm.at[pl.ds(base, SCAN_CH)], srcv[b], semd)

    def scan_wait(b):
        pltpu.make_async_copy(dst_hbm.at[pl.ds(0, SCAN_CH)], dstv[b], semd).wait()
        pltpu.make_async_copy(src_hbm.at[pl.ds(0, SCAN_CH)], srcv[b], semd).wait()

    scan_start(0, 0)

    def chunk_one(ci, b, carry):
        vc, off = carry
        scan_wait(b)

        @pl.when(ci + 1 < E // SCAN_CH)
        def _():
            scan_start(ci + 1, 1 - b)

        spanv = plsc.bitcast(jnp.zeros((L,), jnp.int32) + (hi - lo), jnp.uint32)

        def grp4(q, vcv):
            out = vcv
            for u in range(4):
                g = q * 4 + u
                d = dstv[b][pl.ds(g * L, L)]
                sv = srcv[b][pl.ds(g * L, L)]
                dl = d - lo
                m = plsc.bitcast(dl, jnp.uint32) < spanv
                csum = jnp.cumsum(m.astype(jnp.int32))
                pos = out + csum - 1
                plsc.store_scatter(stage, [pos], sv * SHIFT + dl, mask=m)
                out = out + plsc.all_reduce_population_count(m)
            return out
        vc = lax.fori_loop(0, SCAN_CH // L // 4, grp4, vc)
        vcs = jnp.max(vc)

        def do_flush(args):
            v, o = args
            k = vcs // FLUSH

            def fl(j, oo):
                so = pl.multiple_of(j * FLUSH, 8)
                fo = pl.multiple_of(wid * ECAP + oo, 8)
                pltpu.sync_copy(stage.at[pl.ds(so, FLUSH)],
                                pk_hbm.at[pl.ds(fo, FLUSH)])
                return oo + FLUSH
            o2 = lax.fori_loop(0, k, fl, o)
            rem = vcs - k * FLUSH
            mvbase = k * FLUSH

            def mv(i, _):
                stage[pl.ds(i * L, L)] = stage[pl.ds(mvbase + i * L, L)]
                return 0
            lax.fori_loop(0, (rem + L - 1) // L, mv, 0)
            return (v - k * FLUSH, o2)

        return lax.cond(vcs >= FLUSH, do_flush, lambda a: a, (vc, off))

    def chunk_pair(p, carry):
        for b in range(2):
            carry = chunk_one(p * 2 + b, b, carry)
        return carry

    vc, off = lax.fori_loop(0, E // SCAN_CH // 2, chunk_pair,
                            (jnp.zeros((L,), jnp.int32), jnp.int32(0)))
    fo = pl.multiple_of(wid * ECAP + off, 8)
    pltpu.sync_copy(stage.at[pl.ds(0, FLUSH)], pk_hbm.at[pl.ds(fo, FLUSH)])
    ct = off + jnp.max(vc)
    cbuf[pl.ds(0, L)] = jnp.zeros((L,), jnp.int32) + ct
    pltpu.sync_copy(cbuf.at[pl.ds(0, L)],
                    cnt_hbm.at[pl.ds(pl.multiple_of(wid * L, 8), L)])

    _gather_max_fold(x_hbm, pk_hbm, agg_hbm, pkv, idxv, ldv, rows, agg1d,
                     [sem0, sem1, sem2, sem3], wid, ct)


def _sc_layer2_body(h_hbm, pk_hbm, cnt_hbm,
                    agg_hbm,
                    cntv, pkv, idxv, ldv, rows, agg1d,
                    sem0, sem1, sem2, sem3):
    c = lax.axis_index("c")
    s = lax.axis_index("s")
    wid = c * NS + s
    pltpu.sync_copy(cnt_hbm, cntv)
    ct = jnp.max(cntv[pl.ds(wid * L, L)])
    _gather_max_fold(h_hbm, pk_hbm, agg_hbm, pkv, idxv, ldv, rows, agg1d,
                     [sem0, sem1, sem2, sem3], wid, ct)


def _pair(shape, dtype):
    return [pltpu.VMEM(shape, dtype), pltpu.VMEM(shape, dtype)]


def _sc_layer1(x, src, dst):
    f = pl.kernel(
        _sc_layer1_body,
        out_type=[
            jax.ShapeDtypeStruct((NPAD * D,), jnp.float32),
            jax.ShapeDtypeStruct((NW * ECAP,), jnp.int32),
            jax.ShapeDtypeStruct((NW * L,), jnp.int32),
        ],
        mesh=_mesh(),
        compiler_params=pltpu.CompilerParams(needs_layout_passes=False, disable_bounds_checks=True),
        scratch_types=[
            _pair((SCAN_CH,), jnp.int32),
            _pair((SCAN_CH,), jnp.int32),
            pltpu.VMEM((STAGE,), jnp.int32),
            _pair((GB,), jnp.int32),
            _pair((GB,), jnp.int32),
            _pair((GB,), jnp.int32),
            _pair((GB, D), jnp.float32),
            pltpu.VMEM(((RPW + 1) * D,), jnp.float32),
            pltpu.VMEM((L,), jnp.int32),
            pltpu.SemaphoreType.DMA,
            pltpu.SemaphoreType.DMA,
            pltpu.SemaphoreType.DMA,
            pltpu.SemaphoreType.DMA,
            pltpu.SemaphoreType.DMA,
        ],
    )
    return f(x, src, dst)


def _sc_layer2(h, pk, cnt):
    f = pl.kernel(
        _sc_layer2_body,
        out_type=jax.ShapeDtypeStruct((NPAD * D,), jnp.float32),
        mesh=_mesh(),
        compiler_params=pltpu.CompilerParams(needs_layout_passes=False, disable_bounds_checks=True),
        scratch_types=[
            pltpu.VMEM((NW * L,), jnp.int32),
            _pair((GB,), jnp.int32),
            _pair((GB,), jnp.int32),
            _pair((GB,), jnp.int32),
            _pair((GB, D), jnp.float32),
            pltpu.VMEM(((RPW + 1) * D,), jnp.float32),
            pltpu.SemaphoreType.DMA,
            pltpu.SemaphoreType.DMA,
            pltpu.SemaphoreType.DMA,
            pltpu.SemaphoreType.DMA,
        ],
    )
    return f(h, pk, cnt)


def _lin_body(relu, agg_ref, h_ref, wl_ref, wr_ref, b_ref, o_ref):
    a = agg_ref[...]
    a = jnp.where(a == -jnp.inf, 0.0, a)
    out = lax.dot_general(a, wl_ref[...], (((1,), (1,)), ((), ())),
                          preferred_element_type=jnp.float32)
    out = out + lax.dot_general(h_ref[...], wr_ref[...], (((1,), (1,)), ((), ())),
                                preferred_element_type=jnp.float32)
    out = out + b_ref[...]
    if relu:
        out = jnp.maximum(out, 0.0)
    o_ref[...] = out


def _linear(agg, h, W_l, b_l, W_r, relu):
    BM = 1000
    return pl.pallas_call(
        functools.partial(_lin_body, relu),
        grid=(N // BM,),
        in_specs=[
            pl.BlockSpec((BM, D), lambda i: (i, 0)),
            pl.BlockSpec((BM, D), lambda i: (i, 0)),
            pl.BlockSpec((D, D), lambda i: (0, 0)),
            pl.BlockSpec((D, D), lambda i: (0, 0)),
            pl.BlockSpec((1, D), lambda i: (0, 0)),
        ],
        out_specs=pl.BlockSpec((BM, D), lambda i: (i, 0)),
        out_shape=jax.ShapeDtypeStruct((N, D), jnp.float32),
    )(agg, h, W_l, W_r, b_l)


def kernel(x, edge_index, W1_l, b1_l, W1_r, W2_l, b2_l, W2_r):
    src = edge_index[0].astype(jnp.int32)
    dst = edge_index[1].astype(jnp.int32)
    agg1f, pk, cnt = _sc_layer1(x, src, dst)
    agg1 = agg1f.reshape(NPAD, D)[:N]
    h1 = _linear(agg1, x, W1_l, b1_l.reshape(1, D), W1_r, relu=True)
    agg2f = _sc_layer2(h1, pk, cnt)
    agg2 = agg2f.reshape(NPAD, D)[:N]
    return _linear(agg2, h1, W2_l, b2_l.reshape(1, D), W2_r, relu=False)


# speedup vs baseline: 5.7894x; 1.2702x over previous
import functools

import jax
import jax.numpy as jnp
from jax import lax
from jax.experimental import pallas as pl
from jax.experimental.pallas import tpu as pltpu
from jax.experimental.pallas import tpu_sc as plsc

N = 10000
E = 320000
D = 128
NC, NS, L = 2, 16, 16
NW = NC * NS
RPW = 313
NPAD = NW * RPW
SCAN_CH = 3200
FLUSH = 2048
STAGE = 4096 + 2048
ECAP = E + FLUSH
GB = 128
QD = D // L
SHIFT = 512

_mesh = lambda: plsc.VectorSubcoreMesh(core_axis_name="c", subcore_axis_name="s")


def _gather_max_fold(h_hbm, pk_hbm, agg_hbm, pkv, idxv, ldv, rows, agg1d,
                     sems, wid, ct):
    iota = lax.iota(jnp.int32, L)
    neg = jnp.full((L,), -jnp.inf, dtype=jnp.float32)

    def init_body(i, _):
        for q in range(16):
            agg1d[pl.ds(i * 256 + q * L, L)] = neg
        return 0
    lax.fori_loop(0, (RPW + 1) * D // 256, init_body, 0)

    nch = (ct + GB - 1) // GB

    def idx_start(g, b):
        base = pl.multiple_of(wid * ECAP + g * GB, 8)
        pltpu.async_copy(pk_hbm.at[pl.ds(base, GB)], pkv[b], sems[2 + b])

    def idx_wait_clean(g, b):
        pltpu.make_async_copy(pk_hbm.at[pl.ds(0, GB)], pkv[b],
                              sems[2 + b]).wait()
        for q in range(GB // L):
            w = pkv[b][pl.ds(q * L, L)]
            m = (g * GB + q * L + iota) < ct
            idxv[b][pl.ds(q * L, L)] = jnp.where(m, w // SHIFT, 0)
            ldv[b][pl.ds(q * L, L)] = jnp.where(m, w & (SHIFT - 1), RPW)

    def row_start(b):
        pltpu.async_copy(h_hbm.at[idxv[b]], rows[b], sems[b])

    def row_wait(b):
        pltpu.make_async_copy(h_hbm.at[pl.ds(0, GB)], rows[b], sems[b]).wait()

    def fold_chunk(b):
        def pair_block(e0):
            esp0 = jnp.zeros((L,), jnp.int32) + e0
            esp1 = esp0 + 1
            lds0 = plsc.load_gather(ldv[b], [esp0])
            lds1 = plsc.load_gather(ldv[b], [esp1])
            same = lds0 == lds1
            ab0 = lds0 * D
            ab1 = lds1 * D
            cols = [q * L + iota for q in range(QD)]
            a0s = [ab0 + c for c in cols]
            a1s = [ab1 + c for c in cols]
            r0s = [plsc.load_gather(rows[b], [esp0, c]) for c in cols]
            r1s = [plsc.load_gather(rows[b], [esp1, c]) for c in cols]
            c0s = [plsc.load_gather(agg1d, [a]) for a in a0s]
            c1s = [plsc.load_gather(agg1d, [a]) for a in a1s]
            for q in range(QD):
                comb = jnp.maximum(r0s[q], r1s[q])
                v0 = jnp.where(same, comb, r0s[q])
                v1 = jnp.where(same, comb, r1s[q])
                plsc.store_scatter(agg1d, [a0s[q]], jnp.maximum(c0s[q], v0))
                plsc.store_scatter(agg1d, [a1s[q]], jnp.maximum(c1s[q], v1))

        def fold(p, _):
            pair_block(p * 4)
            pair_block(p * 4 + 2)
            return 0
        lax.fori_loop(0, GB // 4, fold, 0)

    @pl.when(nch > 0)
    def _():
        idx_start(0, 0)
        idx_wait_clean(0, 0)
        row_start(0)

    @pl.when(nch > 1)
    def _():
        idx_start(1, 1)

    def pair(p, _):
        for b in range(2):
            g = p * 2 + b

            @pl.when(g < nch)
            def _():
                row_wait(b)

                @pl.when(g + 1 < nch)
                def _():
                    idx_wait_clean(g + 1, 1 - b)
                    row_start(1 - b)

                @pl.when(g + 2 < nch)
                def _():
                    idx_start(g + 2, b)

                fold_chunk(b)
        return 0
    lax.fori_loop(0, (nch + 1) // 2, pair, 0)

    pltpu.sync_copy(agg1d.at[pl.ds(0, RPW * D)],
                    agg_hbm.at[pl.ds(pl.multiple_of(wid * RPW * D, 8), RPW * D)])


def _sc_layer1_body(x_hbm, src_hbm, dst_hbm,
                    agg_hbm, pk_hbm, cnt_hbm,
                    dstv, srcv, stage, pkv, idxv, ldv, rows, agg1d,
                    cbuf, sem0, sem1, sem2, sem3, semd):
    c = lax.axis_index("c")
    s = lax.axis_index("s")
    wid = c * NS + s
    lo = wid * RPW
    hi = jnp.minimum(lo + RPW, N)
    iota = lax.iota(jnp.int32, L)

    def scan_start(ci, b):
        base = pl.multiple_of(ci * SCAN_CH, 8)
        pltpu.async_copy(dst_hbm.at[pl.ds(base, SCAN_CH)], dstv[b], semd)
        pltpu.async_copy(src_hbm.at[pl.ds(base, SCAN_CH)], srcv[b], semd)

    def scan_wait(b):
        pltpu.make_async_copy(dst_hbm.at[pl.ds(0, SCAN_CH)], dstv[b], semd).wait()
        pltpu.make_async_copy(src_hbm.at[pl.ds(0, SCAN_CH)], srcv[b], semd).wait()

    scan_start(0, 0)

    def chunk_one(ci, b, carry):
        vc, off = carry
        scan_wait(b)

        @pl.when(ci + 1 < E // SCAN_CH)
        def _():
            scan_start(ci + 1, 1 - b)

        spanv = plsc.bitcast(jnp.zeros((L,), jnp.int32) + (hi - lo), jnp.uint32)

        def grp4(q, vcv):
            ds_ = [dstv[b][pl.ds((q * 4 + u) * L, L)] for u in range(4)]
            ss_ = [srcv[b][pl.ds((q * 4 + u) * L, L)] for u in range(4)]
            dls = [d - lo for d in ds_]
            ms = [plsc.bitcast(dl, jnp.uint32) < spanv for dl in dls]
            csums = [jnp.cumsum(m.astype(jnp.int32)) for m in ms]
            pcs = [plsc.all_reduce_population_count(m) for m in ms]
            vals = [ss_[u] * SHIFT + dls[u] for u in range(4)]
            out = vcv
            for u in range(4):
                plsc.store_scatter(stage, [out + csums[u] - 1], vals[u],
                                   mask=ms[u])
                out = out + pcs[u]
            return out
        vc = lax.fori_loop(0, SCAN_CH // L // 4, grp4, vc)
        vcs = jnp.max(vc)

        def do_flush(args):
            v, o = args
            k = vcs // FLUSH

            def fl(j, oo):
                so = pl.multiple_of(j * FLUSH, 8)
                fo = pl.multiple_of(wid * ECAP + oo, 8)
                pltpu.sync_copy(stage.at[pl.ds(so, FLUSH)],
                                pk_hbm.at[pl.ds(fo, FLUSH)])
                return oo + FLUSH
            o2 = lax.fori_loop(0, k, fl, o)
            rem = vcs - k * FLUSH
            mvbase = k * FLUSH

            def mv(i, _):
                stage[pl.ds(i * L, L)] = stage[pl.ds(mvbase + i * L, L)]
                return 0
            lax.fori_loop(0, (rem + L - 1) // L, mv, 0)
            return (v - k * FLUSH, o2)

        return lax.cond(vcs >= FLUSH, do_flush, lambda a: a, (vc, off))

    def chunk_pair(p, carry):
        for b in range(2):
            carry = chunk_one(p * 2 + b, b, carry)
        return carry

    vc, off = lax.fori_loop(0, E // SCAN_CH // 2, chunk_pair,
                            (jnp.zeros((L,), jnp.int32), jnp.int32(0)))
    fo = pl.multiple_of(wid * ECAP + off, 8)
    pltpu.sync_copy(stage.at[pl.ds(0, FLUSH)], pk_hbm.at[pl.ds(fo, FLUSH)])
    ct = off + jnp.max(vc)
    cbuf[pl.ds(0, L)] = jnp.zeros((L,), jnp.int32) + ct
    pltpu.sync_copy(cbuf.at[pl.ds(0, L)],
                    cnt_hbm.at[pl.ds(pl.multiple_of(wid * L, 8), L)])

    _gather_max_fold(x_hbm, pk_hbm, agg_hbm, pkv, idxv, ldv, rows, agg1d,
                     [sem0, sem1, sem2, sem3], wid, ct)


def _sc_layer2_body(h_hbm, pk_hbm, cnt_hbm,
                    agg_hbm,
                    cntv, pkv, idxv, ldv, rows, agg1d,
                    sem0, sem1, sem2, sem3):
    c = lax.axis_index("c")
    s = lax.axis_index("s")
    wid = c * NS + s
    pltpu.sync_copy(cnt_hbm, cntv)
    ct = jnp.max(cntv[pl.ds(wid * L, L)])
    _gather_max_fold(h_hbm, pk_hbm, agg_hbm, pkv, idxv, ldv, rows, agg1d,
                     [sem0, sem1, sem2, sem3], wid, ct)


def _pair(shape, dtype):
    return [pltpu.VMEM(shape, dtype), pltpu.VMEM(shape, dtype)]


def _sc_layer1(x, src, dst):
    f = pl.kernel(
        _sc_layer1_body,
        out_type=[
            jax.ShapeDtypeStruct((NPAD * D,), jnp.float32),
            jax.ShapeDtypeStruct((NW * ECAP,), jnp.int32),
            jax.ShapeDtypeStruct((NW * L,), jnp.int32),
        ],
        mesh=_mesh(),
        compiler_params=pltpu.CompilerParams(needs_layout_passes=False, disable_bounds_checks=True),
        scratch_types=[
            _pair((SCAN_CH,), jnp.int32),
            _pair((SCAN_CH,), jnp.int32),
            pltpu.VMEM((STAGE,), jnp.int32),
            _pair((GB,), jnp.int32),
            _pair((GB,), jnp.int32),
            _pair((GB,), jnp.int32),
            _pair((GB, D), jnp.float32),
            pltpu.VMEM(((RPW + 1) * D,), jnp.float32),
            pltpu.VMEM((L,), jnp.int32),
            pltpu.SemaphoreType.DMA,
            pltpu.SemaphoreType.DMA,
            pltpu.SemaphoreType.DMA,
            pltpu.SemaphoreType.DMA,
            pltpu.SemaphoreType.DMA,
        ],
    )
    return f(x, src, dst)


def _sc_layer2(h, pk, cnt):
    f = pl.kernel(
        _sc_layer2_body,
        out_type=jax.ShapeDtypeStruct((NPAD * D,), jnp.float32),
        mesh=_mesh(),
        compiler_params=pltpu.CompilerParams(needs_layout_passes=False, disable_bounds_checks=True),
        scratch_types=[
            pltpu.VMEM((NW * L,), jnp.int32),
            _pair((GB,), jnp.int32),
            _pair((GB,), jnp.int32),
            _pair((GB,), jnp.int32),
            _pair((GB, D), jnp.float32),
            pltpu.VMEM(((RPW + 1) * D,), jnp.float32),
            pltpu.SemaphoreType.DMA,
            pltpu.SemaphoreType.DMA,
            pltpu.SemaphoreType.DMA,
            pltpu.SemaphoreType.DMA,
        ],
    )
    return f(h, pk, cnt)


def _lin_body(relu, agg_ref, h_ref, wl_ref, wr_ref, b_ref, o_ref):
    a = agg_ref[...]
    a = jnp.where(a == -jnp.inf, 0.0, a)
    out = lax.dot_general(a, wl_ref[...], (((1,), (1,)), ((), ())),
                          preferred_element_type=jnp.float32)
    out = out + lax.dot_general(h_ref[...], wr_ref[...], (((1,), (1,)), ((), ())),
                                preferred_element_type=jnp.float32)
    out = out + b_ref[...]
    if relu:
        out = jnp.maximum(out, 0.0)
    o_ref[...] = out


def _linear(agg, h, W_l, b_l, W_r, relu):
    BM = 1000
    return pl.pallas_call(
        functools.partial(_lin_body, relu),
        grid=(N // BM,),
        in_specs=[
            pl.BlockSpec((BM, D), lambda i: (i, 0)),
            pl.BlockSpec((BM, D), lambda i: (i, 0)),
            pl.BlockSpec((D, D), lambda i: (0, 0)),
            pl.BlockSpec((D, D), lambda i: (0, 0)),
            pl.BlockSpec((1, D), lambda i: (0, 0)),
        ],
        out_specs=pl.BlockSpec((BM, D), lambda i: (i, 0)),
        out_shape=jax.ShapeDtypeStruct((N, D), jnp.float32),
    )(agg, h, W_l, W_r, b_l)


def kernel(x, edge_index, W1_l, b1_l, W1_r, W2_l, b2_l, W2_r):
    src = edge_index[0].astype(jnp.int32)
    dst = edge_index[1].astype(jnp.int32)
    agg1f, pk, cnt = _sc_layer1(x, src, dst)
    agg1 = agg1f.reshape(NPAD, D)[:N]
    h1 = _linear(agg1, x, W1_l, b1_l.reshape(1, D), W1_r, relu=True)
    agg2f = _sc_layer2(h1, pk, cnt)
    agg2 = agg2f.reshape(NPAD, D)[:N]
    return _linear(agg2, h1, W2_l, b2_l.reshape(1, D), W2_r, relu=False)


# speedup vs baseline: 6.1890x; 1.0690x over previous
import functools

import jax
import jax.numpy as jnp
import numpy as np
from jax import lax
from jax.experimental import pallas as pl
from jax.experimental.pallas import tpu as pltpu
from jax.experimental.pallas import tpu_sc as plsc

N = 10000
E = 320000
D = 128
NC, NS, L = 2, 16, 16
NW = NC * NS
RPW = 313
NPAD = NW * RPW
SCAN_CH = 3200
FLUSH = 2048
STAGE = 4096 + 2048
ECAP = E + FLUSH
GB = 128
QD = D // L
SHIFT = 512

_mesh = lambda: plsc.VectorSubcoreMesh(core_axis_name="c", subcore_axis_name="s")


def _gather_max_fold(h_hbm, pk_hbm, agg_hbm, pkv, idxv, ldv, rows, agg1d,
                     sems, wid, ct):
    iota = lax.iota(jnp.int32, L)
    neg = jnp.full((L,), -jnp.inf, dtype=jnp.float32)

    def init_body(i, _):
        for q in range(16):
            agg1d[pl.ds(i * 256 + q * L, L)] = neg
        return 0
    lax.fori_loop(0, (RPW + 1) * D // 256, init_body, 0)

    nch = (ct + GB - 1) // GB

    def idx_start(g, b):
        base = pl.multiple_of(wid * ECAP + g * GB, 8)
        pltpu.async_copy(pk_hbm.at[pl.ds(base, GB)], pkv[b], sems[2 + b])

    def idx_wait_clean(g, b):
        pltpu.make_async_copy(pk_hbm.at[pl.ds(0, GB)], pkv[b],
                              sems[2 + b]).wait()
        for q in range(GB // L):
            w = pkv[b][pl.ds(q * L, L)]
            m = (g * GB + q * L + iota) < ct
            idxv[b][pl.ds(q * L, L)] = jnp.where(m, w // SHIFT, 0)
            ldv[b][pl.ds(q * L, L)] = jnp.where(m, w & (SHIFT - 1), RPW)

    def row_start(b):
        pltpu.async_copy(h_hbm.at[idxv[b]], rows[b], sems[b])

    def row_wait(b):
        pltpu.make_async_copy(h_hbm.at[pl.ds(0, GB)], rows[b], sems[b]).wait()

    def fold_chunk(b):
        def pair_block(e0):
            esp0 = jnp.zeros((L,), jnp.int32) + e0
            esp1 = esp0 + 1
            lds0 = plsc.load_gather(ldv[b], [esp0])
            lds1 = plsc.load_gather(ldv[b], [esp1])
            same = lds0 == lds1
            ab0 = lds0 * D
            ab1 = lds1 * D
            cols = [q * L + iota for q in range(QD)]
            a0s = [ab0 + c for c in cols]
            a1s = [ab1 + c for c in cols]
            w0s = [rows[b][e0, pl.ds(q2 * L, L)]
                   for q2 in range(QD // 2)]
            w1s = [rows[b][e0 + 1, pl.ds(q2 * L, L)]
                   for q2 in range(QD // 2)]
            c0s = [plsc.load_gather(agg1d, [a]) for a in a0s]
            c1s = [plsc.load_gather(agg1d, [a]) for a in a1s]
            r0s, r1s = [], []
            for q2 in range(QD // 2):
                ra, rb = plsc.unpack(plsc.bitcast(w0s[q2], jnp.bfloat16),
                                     format=plsc.PackFormat.INTERLEAVED,
                                     preferred_element_type=jnp.float32)
                r0s += [ra, rb]
                ra, rb = plsc.unpack(plsc.bitcast(w1s[q2], jnp.bfloat16),
                                     format=plsc.PackFormat.INTERLEAVED,
                                     preferred_element_type=jnp.float32)
                r1s += [ra, rb]
            for q in range(QD):
                comb = jnp.maximum(r0s[q], r1s[q])
                v0 = jnp.where(same, comb, r0s[q])
                v1 = jnp.where(same, comb, r1s[q])
                plsc.store_scatter(agg1d, [a0s[q]], jnp.maximum(c0s[q], v0))
                plsc.store_scatter(agg1d, [a1s[q]], jnp.maximum(c1s[q], v1))

        def fold(p, _):
            pair_block(p * 4)
            pair_block(p * 4 + 2)
            return 0
        lax.fori_loop(0, GB // 4, fold, 0)

    @pl.when(nch > 0)
    def _():
        idx_start(0, 0)
        idx_wait_clean(0, 0)
        row_start(0)

    @pl.when(nch > 1)
    def _():
        idx_start(1, 1)

    def pair(p, _):
        for b in range(2):
            g = p * 2 + b

            @pl.when(g < nch)
            def _():
                row_wait(b)

                @pl.when(g + 1 < nch)
                def _():
                    idx_wait_clean(g + 1, 1 - b)
                    row_start(1 - b)

                @pl.when(g + 2 < nch)
                def _():
                    idx_start(g + 2, b)

                fold_chunk(b)
        return 0
    lax.fori_loop(0, (nch + 1) // 2, pair, 0)

    pltpu.sync_copy(agg1d.at[pl.ds(0, RPW * D)],
                    agg_hbm.at[pl.ds(pl.multiple_of(wid * RPW * D, 8), RPW * D)])


def _sc_layer1_body(x_hbm, src_hbm, dst_hbm,
                    agg_hbm, pk_hbm, cnt_hbm,
                    dstv, srcv, stage, pkv, idxv, ldv, rows, agg1d,
                    cbuf, sem0, sem1, sem2, sem3, semd):
    c = lax.axis_index("c")
    s = lax.axis_index("s")
    wid = c * NS + s
    lo = wid * RPW
    hi = jnp.minimum(lo + RPW, N)
    iota = lax.iota(jnp.int32, L)

    def scan_start(ci, b):
        base = pl.multiple_of(ci * SCAN_CH, 8)
        pltpu.async_copy(dst_hbm.at[pl.ds(base, SCAN_CH)], dstv[b], semd)
        pltpu.async_copy(src_hbm.at[pl.ds(base, SCAN_CH)], srcv[b], semd)

    def scan_wait(b):
        pltpu.make_async_copy(dst_hbm.at[pl.ds(0, SCAN_CH)], dstv[b], semd).wait()
        pltpu.make_async_copy(src_hbm.at[pl.ds(0, SCAN_CH)], srcv[b], semd).wait()

    scan_start(0, 0)

    def chunk_one(ci, b, carry):
        vc, off = carry
        scan_wait(b)

        @pl.when(ci + 1 < E // SCAN_CH)
        def _():
            scan_start(ci + 1, 1 - b)

        spanv = plsc.bitcast(jnp.zeros((L,), jnp.int32) + (hi - lo), jnp.uint32)

        def grp4(q, vcv):
            ds_ = [dstv[b][pl.ds((q * 4 + u) * L, L)] for u in range(4)]
            ss_ = [srcv[b][pl.ds((q * 4 + u) * L, L)] for u in range(4)]
            dls = [d - lo for d in ds_]
            ms = [plsc.bitcast(dl, jnp.uint32) < spanv for dl in dls]
            csums = [jnp.cumsum(m.astype(jnp.int32)) for m in ms]
            pcs = [plsc.all_reduce_population_count(m) for m in ms]
            vals = [ss_[u] * SHIFT + dls[u] for u in range(4)]
            out = vcv
            for u in range(4):
                plsc.store_scatter(stage, [out + csums[u] - 1], vals[u],
                                   mask=ms[u])
                out = out + pcs[u]
            return out
        vc = lax.fori_loop(0, SCAN_CH // L // 4, grp4, vc)
        vcs = jnp.max(vc)

        def do_flush(args):
            v, o = args
            k = vcs // FLUSH

            def fl(j, oo):
                so = pl.multiple_of(j * FLUSH, 8)
                fo = pl.multiple_of(wid * ECAP + oo, 8)
                pltpu.sync_copy(stage.at[pl.ds(so, FLUSH)],
                                pk_hbm.at[pl.ds(fo, FLUSH)])
                return oo + FLUSH
            o2 = lax.fori_loop(0, k, fl, o)
            rem = vcs - k * FLUSH
            mvbase = k * FLUSH

            def mv(i, _):
                stage[pl.ds(i * L, L)] = stage[pl.ds(mvbase + i * L, L)]
                return 0
            lax.fori_loop(0, (rem + L - 1) // L, mv, 0)
            return (v - k * FLUSH, o2)

        return lax.cond(vcs >= FLUSH, do_flush, lambda a: a, (vc, off))

    def chunk_pair(p, carry):
        for b in range(2):
            carry = chunk_one(p * 2 + b, b, carry)
        return carry

    vc, off = lax.fori_loop(0, E // SCAN_CH // 2, chunk_pair,
                            (jnp.zeros((L,), jnp.int32), jnp.int32(0)))
    fo = pl.multiple_of(wid * ECAP + off, 8)
    pltpu.sync_copy(stage.at[pl.ds(0, FLUSH)], pk_hbm.at[pl.ds(fo, FLUSH)])
    ct = off + jnp.max(vc)
    cbuf[pl.ds(0, L)] = jnp.zeros((L,), jnp.int32) + ct
    pltpu.sync_copy(cbuf.at[pl.ds(0, L)],
                    cnt_hbm.at[pl.ds(pl.multiple_of(wid * L, 8), L)])

    _gather_max_fold(x_hbm, pk_hbm, agg_hbm, pkv, idxv, ldv, rows, agg1d,
                     [sem0, sem1, sem2, sem3], wid, ct)


def _sc_layer2_body(h_hbm, pk_hbm, cnt_hbm,
                    agg_hbm,
                    cntv, pkv, idxv, ldv, rows, agg1d,
                    sem0, sem1, sem2, sem3):
    c = lax.axis_index("c")
    s = lax.axis_index("s")
    wid = c * NS + s
    pltpu.sync_copy(cnt_hbm, cntv)
    ct = jnp.max(cntv[pl.ds(wid * L, L)])
    _gather_max_fold(h_hbm, pk_hbm, agg_hbm, pkv, idxv, ldv, rows, agg1d,
                     [sem0, sem1, sem2, sem3], wid, ct)


def _pair(shape, dtype):
    return [pltpu.VMEM(shape, dtype), pltpu.VMEM(shape, dtype)]


def _sc_layer1(x, src, dst):
    f = pl.kernel(
        _sc_layer1_body,
        out_type=[
            jax.ShapeDtypeStruct((NPAD * D,), jnp.float32),
            jax.ShapeDtypeStruct((NW * ECAP,), jnp.int32),
            jax.ShapeDtypeStruct((NW * L,), jnp.int32),
        ],
        mesh=_mesh(),
        compiler_params=pltpu.CompilerParams(needs_layout_passes=False, disable_bounds_checks=True, use_tc_tiling_on_sc=False),
        scratch_types=[
            _pair((SCAN_CH,), jnp.int32),
            _pair((SCAN_CH,), jnp.int32),
            pltpu.VMEM((STAGE,), jnp.int32),
            _pair((GB,), jnp.int32),
            _pair((GB,), jnp.int32),
            _pair((GB,), jnp.int32),
            _pair((GB, D // 2), jnp.int32),
            pltpu.VMEM(((RPW + 1) * D,), jnp.float32),
            pltpu.VMEM((L,), jnp.int32),
            pltpu.SemaphoreType.DMA,
            pltpu.SemaphoreType.DMA,
            pltpu.SemaphoreType.DMA,
            pltpu.SemaphoreType.DMA,
            pltpu.SemaphoreType.DMA,
        ],
    )
    return f(x, src, dst)


def _sc_layer2(h, pk, cnt):
    f = pl.kernel(
        _sc_layer2_body,
        out_type=jax.ShapeDtypeStruct((NPAD * D,), jnp.float32),
        mesh=_mesh(),
        compiler_params=pltpu.CompilerParams(needs_layout_passes=False, disable_bounds_checks=True, use_tc_tiling_on_sc=False),
        scratch_types=[
            pltpu.VMEM((NW * L,), jnp.int32),
            _pair((GB,), jnp.int32),
            _pair((GB,), jnp.int32),
            _pair((GB,), jnp.int32),
            _pair((GB, D // 2), jnp.int32),
            pltpu.VMEM(((RPW + 1) * D,), jnp.float32),
            pltpu.SemaphoreType.DMA,
            pltpu.SemaphoreType.DMA,
            pltpu.SemaphoreType.DMA,
            pltpu.SemaphoreType.DMA,
        ],
    )
    return f(h, pk, cnt)


def _lin_body(relu, agg_ref, h_ref, wl_ref, wr_ref, b_ref, o_ref):
    a = agg_ref[...]
    a = jnp.where(a == -jnp.inf, 0.0, a)
    out = lax.dot_general(a, wl_ref[...], (((1,), (1,)), ((), ())),
                          preferred_element_type=jnp.float32)
    out = out + lax.dot_general(h_ref[...], wr_ref[...], (((1,), (1,)), ((), ())),
                                preferred_element_type=jnp.float32)
    out = out + b_ref[...]
    if relu:
        out = jnp.maximum(out, 0.0)
    o_ref[...] = out


def _linear(agg, h, W_l, b_l, W_r, relu):
    BM = 1000
    return pl.pallas_call(
        functools.partial(_lin_body, relu),
        grid=(N // BM,),
        in_specs=[
            pl.BlockSpec((BM, D), lambda i: (i, 0)),
            pl.BlockSpec((BM, D), lambda i: (i, 0)),
            pl.BlockSpec((D, D), lambda i: (0, 0)),
            pl.BlockSpec((D, D), lambda i: (0, 0)),
            pl.BlockSpec((1, D), lambda i: (0, 0)),
        ],
        out_specs=pl.BlockSpec((BM, D), lambda i: (i, 0)),
        out_shape=jax.ShapeDtypeStruct((N, D), jnp.float32),
    )(agg, h, W_l, W_r, b_l)


def _to_words(a):
    bf = a.astype(jnp.bfloat16)
    return lax.bitcast_convert_type(bf.reshape(N, D // 2, 2), jnp.int32)


_PERM = np.empty(D, np.int32)
for _q2 in range(D // 32):
    for _k in range(16):
        _PERM[_q2 * 32 + _k] = _q2 * 32 + 2 * _k
        _PERM[_q2 * 32 + 16 + _k] = _q2 * 32 + 2 * _k + 1


def kernel(x, edge_index, W1_l, b1_l, W1_r, W2_l, b2_l, W2_r):
    src = edge_index[0].astype(jnp.int32)
    dst = edge_index[1].astype(jnp.int32)
    agg1f, pk, cnt = _sc_layer1(_to_words(x), src, dst)
    agg1 = agg1f.reshape(NPAD, D)[:N]
    h1 = _linear(agg1, x, W1_l[:, _PERM], b1_l.reshape(1, D), W1_r, relu=True)
    agg2f = _sc_layer2(_to_words(h1), pk, cnt)
    agg2 = agg2f.reshape(NPAD, D)[:N]
    return _linear(agg2, h1, W2_l[:, _PERM], b2_l.reshape(1, D), W2_r,
                   relu=False)
